# Initial kernel scaffold; baseline (speedup 1.0000x reference)
import jax
import jax.numpy as jnp
from jax.experimental import pallas as pl


def kernel(x, edge_index, W1, b1, W2, b2):
    raise NotImplementedError("write your pallas kernel here")


# speedup vs baseline: 1.1868x; 1.1868x over previous
import jax
import jax.numpy as jnp
from jax.experimental import pallas as pl


def _bias_add_kernel(h_ref, b_ref, o_ref):
    o_ref[...] = h_ref[...] + b_ref[...]


def _layer(x, src, dst, W, b, n, dinv):
    norm = dinv[src] * dinv[dst]
    xw = x @ W
    msg = xw[src] * norm[:, None]
    out = jax.ops.segment_sum(msg, dst, num_segments=n)
    out = out + dinv[:, None] * dinv[:, None] * xw
    return pl.pallas_call(
        _bias_add_kernel,
        out_shape=jax.ShapeDtypeStruct(out.shape, out.dtype),
    )(out, jnp.broadcast_to(b, out.shape))


def kernel(x, edge_index, W1, b1, W2, b2):
    n = x.shape[0]
    src = edge_index[0]
    dst = edge_index[1]
    ones = jnp.ones(src.shape[0], dtype=x.dtype)
    deg = jax.ops.segment_sum(ones, dst, num_segments=n) + 1.0
    dinv = jax.lax.rsqrt(deg)
    h = _layer(x, src, dst, W1, b1, n, dinv)
    h = _layer(h, src, dst, W2, b2, n, dinv)
    return h


# speedup vs baseline: 13.0691x; 11.0119x over previous
import functools

import jax
import jax.numpy as jnp
from jax import lax
from jax.experimental import pallas as pl
from jax.experimental.pallas import tpu as pltpu
from jax.experimental.pallas import tpu_sc as plsc

_N = 10000
_D = 128
_E = 320000
_NC = 2
_NS = 16
_EPT = _E // (_NC * _NS)
_CH = 80
_NCHUNK = _EPT // _CH
_RPT = _N // _NS

_MESH = plsc.VectorSubcoreMesh(
    core_axis_name="c", subcore_axis_name="s", num_cores=_NC, num_subcores=_NS
)


@functools.partial(
    pl.kernel,
    out_type=jax.ShapeDtypeStruct((_NC * _N,), jnp.float32),
    mesh=_MESH,
    scratch_types=[
        pltpu.VMEM((_CH,), jnp.int32),
        pltpu.VMEM((_CH,), jnp.float32),
        pltpu.VMEM((1008,), jnp.float32),
        pltpu.VMEM_SHARED((_N,), jnp.float32),
    ],
)
def _deg_kernel(dst_hbm, deg_hbm, idx_v, ones_v, stg_v, deg_sh):
    c = lax.axis_index("c")
    s = lax.axis_index("s")
    one16 = jnp.full((16,), 1.0, dtype=jnp.float32)
    zero16 = jnp.zeros((16,), dtype=jnp.float32)
    for i in range(_CH // 16):
        ones_v[pl.ds(i * 16, 16)] = one16

    def fill0(i, _):
        stg_v[pl.ds(i * 16, 16)] = zero16
        return ()

    lax.fori_loop(0, 1008 // 16, fill0, ())

    @pl.when(s < 10)
    def _():
        pltpu.sync_copy(stg_v.at[pl.ds(0, 1000)],
                        deg_sh.at[pl.ds(s * 1000, 1000)])

    plsc.subcore_barrier()
    base = (c * _NS + s) * _EPT

    def body(j, _):
        pltpu.sync_copy(dst_hbm.at[pl.ds(base + j * _CH, _CH)], idx_v)
        pltpu.sync_copy(ones_v, deg_sh.at[idx_v], add=True)
        return ()

    lax.fori_loop(0, _NCHUNK, body, ())
    plsc.subcore_barrier()
    @pl.when(s < 10)
    def _():
        pltpu.sync_copy(deg_sh.at[pl.ds(s * 1000, 1000)],
                        stg_v.at[pl.ds(0, 1000)])
        pltpu.sync_copy(stg_v.at[pl.ds(0, 1000)],
                        deg_hbm.at[pl.ds(c * _N + s * 1000, 1000)])


@functools.partial(
    pl.kernel,
    out_type=jax.ShapeDtypeStruct((_NC * _N, _D), jnp.float32),
    mesh=_MESH,
    scratch_types=[
        pltpu.VMEM((_CH,), jnp.int32),
        pltpu.VMEM((_CH,), jnp.int32),
        pltpu.VMEM((_CH, _D), jnp.float32),
        pltpu.VMEM((200, _D), jnp.float32),
        pltpu.VMEM_SHARED((_N, _D), jnp.float32),
    ],
)
def _edge_acc_kernel(y_hbm, src_hbm, dst_hbm, acc_hbm,
                     idxs_v, idxd_v, rows_v, stg_v, acc_sh):
    c = lax.axis_index("c")
    s = lax.axis_index("s")
    zero16 = jnp.zeros((16,), dtype=jnp.float32)

    def fill0(j, _):
        for i in range(_D // 16):
            stg_v[j, pl.ds(i * 16, 16)] = zero16
        return ()

    lax.fori_loop(0, 200, fill0, ())

    @pl.when(s < 10)
    def _():
        for k in range(5):
            pltpu.sync_copy(stg_v,
                            acc_sh.at[pl.ds(s * 1000 + k * 200, 200), :])

    plsc.subcore_barrier()
    base = (c * _NS + s) * _EPT

    def body(j, _):
        off = base + j * _CH
        pltpu.sync_copy(src_hbm.at[pl.ds(off, _CH)], idxs_v)
        pltpu.sync_copy(dst_hbm.at[pl.ds(off, _CH)], idxd_v)
        pltpu.sync_copy(y_hbm.at[idxs_v], rows_v)
        pltpu.sync_copy(rows_v, acc_sh.at[idxd_v], add=True)
        return ()

    lax.fori_loop(0, _NCHUNK, body, ())
    plsc.subcore_barrier()
    @pl.when(s < 10)
    def _():
        for k in range(5):
            pltpu.sync_copy(acc_sh.at[pl.ds(s * 1000 + k * 200, 200), :],
                            stg_v)
            pltpu.sync_copy(stg_v,
                            acc_hbm.at[pl.ds(c * _N + s * 1000 + k * 200, 200), :])


def _tc_head_body(degt_ref, x_ref, w_ref, dinv_ref, y_ref):
    deg = degt_ref[:, 0:1] + degt_ref[:, 1:2] + 1.0
    dinv = lax.rsqrt(deg)
    dinv_ref[...] = dinv
    xw = jnp.dot(x_ref[...], w_ref[...], preferred_element_type=jnp.float32)
    y_ref[...] = xw * dinv


def _tc_mid_body(acc_ref, y_ref, dinv_ref, w_ref, b_ref, y2_ref):
    dinv = dinv_ref[...]
    h = dinv * (acc_ref[0:_N, :] + acc_ref[_N:, :] + y_ref[...]) + b_ref[...]
    hw = jnp.dot(h, w_ref[...], preferred_element_type=jnp.float32)
    y2_ref[...] = hw * dinv


def _tc_tail_body(acc_ref, y_ref, dinv_ref, b_ref, o_ref):
    dinv = dinv_ref[...]
    o_ref[...] = dinv * (acc_ref[0:_N, :] + acc_ref[_N:, :] + y_ref[...]) + b_ref[...]


def kernel(x, edge_index, W1, b1, W2, b2):
    src = edge_index[0]
    dst = edge_index[1]

    degp = _deg_kernel(dst)
    degt = jnp.transpose(degp.reshape(_NC, _N))

    dinv, y1 = pl.pallas_call(
        _tc_head_body,
        out_shape=(
            jax.ShapeDtypeStruct((_N, 1), jnp.float32),
            jax.ShapeDtypeStruct((_N, _D), jnp.float32),
        ),
    )(degt, x, W1)

    acc1 = _edge_acc_kernel(y1, src, dst)

    y2 = pl.pallas_call(
        _tc_mid_body,
        out_shape=jax.ShapeDtypeStruct((_N, _D), jnp.float32),
    )(acc1, y1, dinv, W2, jnp.broadcast_to(b1, (_N, _D)))

    acc2 = _edge_acc_kernel(y2, src, dst)

    out = pl.pallas_call(
        _tc_tail_body,
        out_shape=jax.ShapeDtypeStruct((_N, _D), jnp.float32),
    )(acc2, y2, dinv, jnp.broadcast_to(b2, (_N, _D)))
    return out


# speedup vs baseline: 22.4992x; 1.7216x over previous
import functools

import jax
import jax.numpy as jnp
from jax import lax
from jax.experimental import pallas as pl
from jax.experimental.pallas import tpu as pltpu
from jax.experimental.pallas import tpu_sc as plsc

_N = 10000
_D = 128
_E = 320000
_NC = 2
_NS = 16
_EPT = _E // (_NC * _NS)
_CH = 80
_NCHUNK = _EPT // _CH
_RPT = _N // _NS

_MESH = plsc.VectorSubcoreMesh(
    core_axis_name="c", subcore_axis_name="s", num_cores=_NC, num_subcores=_NS
)


@functools.partial(
    pl.kernel,
    out_type=jax.ShapeDtypeStruct((_NC * _N,), jnp.float32),
    mesh=_MESH,
    scratch_types=[
        pltpu.VMEM((_CH,), jnp.int32),
        pltpu.VMEM((_CH,), jnp.float32),
        pltpu.VMEM((1008,), jnp.float32),
        pltpu.VMEM_SHARED((_N,), jnp.float32),
    ],
)
def _deg_kernel(dst_hbm, deg_hbm, idx_v, ones_v, stg_v, deg_sh):
    c = lax.axis_index("c")
    s = lax.axis_index("s")
    one16 = jnp.full((16,), 1.0, dtype=jnp.float32)
    zero16 = jnp.zeros((16,), dtype=jnp.float32)
    for i in range(_CH // 16):
        ones_v[pl.ds(i * 16, 16)] = one16

    def fill0(i, _):
        stg_v[pl.ds(i * 16, 16)] = zero16
        return ()

    lax.fori_loop(0, 1008 // 16, fill0, ())

    @pl.when(s < 10)
    def _():
        pltpu.sync_copy(stg_v.at[pl.ds(0, 1000)],
                        deg_sh.at[pl.ds(s * 1000, 1000)])

    plsc.subcore_barrier()
    base = (c * _NS + s) * _EPT

    def body(j, _):
        pltpu.sync_copy(dst_hbm.at[pl.ds(base + j * _CH, _CH)], idx_v)
        pltpu.sync_copy(ones_v, deg_sh.at[idx_v], add=True)
        return ()

    lax.fori_loop(0, _NCHUNK, body, ())
    plsc.subcore_barrier()
    @pl.when(s < 10)
    def _():
        pltpu.sync_copy(deg_sh.at[pl.ds(s * 1000, 1000)],
                        stg_v.at[pl.ds(0, 1000)])
        pltpu.sync_copy(stg_v.at[pl.ds(0, 1000)],
                        deg_hbm.at[pl.ds(c * _N + s * 1000, 1000)])


@functools.partial(
    pl.kernel,
    out_type=jax.ShapeDtypeStruct((_NC * _N, _D), jnp.float32),
    mesh=_MESH,
    scratch_types=[
        pltpu.VMEM((_EPT,), jnp.int32),
        pltpu.VMEM((_CH,), jnp.int32),
        pltpu.VMEM((_CH,), jnp.int32),
        pltpu.VMEM((_CH, _D), jnp.float32),
        pltpu.VMEM((_CH, _D), jnp.float32),
        pltpu.VMEM_SHARED((_N, _D), jnp.float32),
        pltpu.SemaphoreType.DMA,
        pltpu.SemaphoreType.DMA,
        pltpu.SemaphoreType.DMA,
        pltpu.SemaphoreType.DMA,
        pltpu.SemaphoreType.DMA,
        pltpu.SemaphoreType.DMA,
    ],
)
def _edge_acc_kernel(y_hbm, src_hbm, dst_hbm, acc_hbm, sidx,
                     idxd0, idxd1, rows0, rows1, acc_sh,
                     gsem0, gsem1, ssem0, ssem1, isem0, isem1):
    c = lax.axis_index("c")
    s = lax.axis_index("s")
    base = (c * _NS + s) * _EPT
    pltpu.sync_copy(src_hbm.at[pl.ds(base, _EPT)], sidx)
    zero16 = jnp.zeros((16,), dtype=jnp.float32)

    def fill0(j, _):
        for i in range(_D // 16):
            rows0[j, pl.ds(i * 16, 16)] = zero16
        return ()

    lax.fori_loop(0, _CH, fill0, ())

    for k in range(8):
        @pl.when(s + 16 * k < _N // _CH)
        def _():
            pltpu.sync_copy(rows0,
                            acc_sh.at[pl.ds((s + 16 * k) * _CH, _CH), :])

    plsc.subcore_barrier()

    def g_start(ch, rbuf, sem):
        pltpu.async_copy(y_hbm.at[sidx.at[pl.ds(ch * _CH, _CH)]], rbuf, sem)

    def g_wait(rbuf, sem):
        pltpu.make_async_copy(y_hbm.at[pl.ds(0, _CH), :], rbuf, sem).wait()

    def sc_start(rbuf, ibuf, sem):
        pltpu.async_copy(rbuf, acc_sh.at[ibuf], sem, add=True)

    def sc_wait(rbuf, sem):
        pltpu.make_async_copy(rbuf, acc_sh.at[pl.ds(0, _CH), :], sem).wait()

    def i_start(ch, ibuf, sem):
        pltpu.async_copy(dst_hbm.at[pl.ds(base + ch * _CH, _CH)], ibuf, sem)

    def i_wait(ibuf, sem):
        pltpu.make_async_copy(dst_hbm.at[pl.ds(0, _CH)], ibuf, sem).wait()

    i_start(0, idxd0, isem0)
    i_start(1, idxd1, isem1)
    g_start(0, rows0, gsem0)
    g_start(1, rows1, gsem1)

    def body(j, _):
        e = 2 * j
        g_wait(rows0, gsem0)
        i_wait(idxd0, isem0)
        sc_start(rows0, idxd0, ssem0)
        g_wait(rows1, gsem1)
        i_wait(idxd1, isem1)
        sc_start(rows1, idxd1, ssem1)
        sc_wait(rows0, ssem0)
        i_start(e + 2, idxd0, isem0)
        g_start(e + 2, rows0, gsem0)
        sc_wait(rows1, ssem1)

        @pl.when(e + 3 < _NCHUNK)
        def _():
            i_start(e + 3, idxd1, isem1)
            g_start(e + 3, rows1, gsem1)

        return ()

    lax.fori_loop(0, (_NCHUNK - 1) // 2, body, ())
    g_wait(rows0, gsem0)
    i_wait(idxd0, isem0)
    pltpu.sync_copy(rows0, acc_sh.at[idxd0], add=True)
    plsc.subcore_barrier()
    for k in range(8):
        @pl.when(s + 16 * k < _N // _CH)
        def _():
            off = (s + 16 * k) * _CH
            pltpu.sync_copy(acc_sh.at[pl.ds(off, _CH), :], rows0)
            pltpu.sync_copy(rows0, acc_hbm.at[pl.ds(c * _N + off, _CH), :])


def _tc_head_body(degt_ref, x_ref, w_ref, dinv_ref, y_ref):
    deg = degt_ref[:, 0:1] + degt_ref[:, 1:2] + 1.0
    dinv = lax.rsqrt(deg)
    dinv_ref[...] = dinv
    xw = jnp.dot(x_ref[...], w_ref[...], preferred_element_type=jnp.float32)
    y_ref[...] = xw * dinv


def _tc_mid_body(acc_ref, y_ref, dinv_ref, w_ref, b_ref, y2_ref):
    dinv = dinv_ref[...]
    h = dinv * (acc_ref[0:_N, :] + acc_ref[_N:, :] + y_ref[...]) + b_ref[...]
    hw = jnp.dot(h, w_ref[...], preferred_element_type=jnp.float32)
    y2_ref[...] = hw * dinv


def _tc_tail_body(acc_ref, y_ref, dinv_ref, b_ref, o_ref):
    dinv = dinv_ref[...]
    o_ref[...] = dinv * (acc_ref[0:_N, :] + acc_ref[_N:, :] + y_ref[...]) + b_ref[...]


def kernel(x, edge_index, W1, b1, W2, b2):
    src = edge_index[0]
    dst = edge_index[1]

    degp = _deg_kernel(dst)
    degt = jnp.transpose(degp.reshape(_NC, _N))

    dinv, y1 = pl.pallas_call(
        _tc_head_body,
        out_shape=(
            jax.ShapeDtypeStruct((_N, 1), jnp.float32),
            jax.ShapeDtypeStruct((_N, _D), jnp.float32),
        ),
    )(degt, x, W1)

    acc1 = _edge_acc_kernel(y1, src, dst)

    y2 = pl.pallas_call(
        _tc_mid_body,
        out_shape=jax.ShapeDtypeStruct((_N, _D), jnp.float32),
    )(acc1, y1, dinv, W2, jnp.broadcast_to(b1, (_N, _D)))

    acc2 = _edge_acc_kernel(y2, src, dst)

    out = pl.pallas_call(
        _tc_tail_body,
        out_shape=jax.ShapeDtypeStruct((_N, _D), jnp.float32),
    )(acc2, y2, dinv, jnp.broadcast_to(b2, (_N, _D)))
    return out


# speedup vs baseline: 25.1276x; 1.1168x over previous
import functools

import jax
import jax.numpy as jnp
from jax import lax
from jax.experimental import pallas as pl
from jax.experimental.pallas import tpu as pltpu
from jax.experimental.pallas import tpu_sc as plsc

_N = 10000
_D = 128
_E = 320000
_NC = 2
_NS = 16
_EPT = _E // (_NC * _NS)
_CH = 80
_NCHUNK = _EPT // _CH
_RPT = _N // _NS

_MESH = plsc.VectorSubcoreMesh(
    core_axis_name="c", subcore_axis_name="s", num_cores=_NC, num_subcores=_NS
)


@functools.partial(
    pl.kernel,
    out_type=jax.ShapeDtypeStruct((_NC * _N,), jnp.float32),
    mesh=_MESH,
    scratch_types=[
        pltpu.VMEM((_CH,), jnp.int32),
        pltpu.VMEM((_CH,), jnp.int32),
        pltpu.VMEM((_CH,), jnp.int32),
        pltpu.VMEM((_CH,), jnp.int32),
        pltpu.VMEM((_CH,), jnp.float32),
        pltpu.VMEM((1008,), jnp.float32),
        pltpu.VMEM_SHARED((_N,), jnp.float32),
        pltpu.SemaphoreType.DMA,
        pltpu.SemaphoreType.DMA,
        pltpu.SemaphoreType.DMA,
        pltpu.SemaphoreType.DMA,
        pltpu.SemaphoreType.DMA,
        pltpu.SemaphoreType.DMA,
        pltpu.SemaphoreType.DMA,
        pltpu.SemaphoreType.DMA,
    ],
)
def _deg_kernel(dst_hbm, deg_hbm, i0, i1, i2, i3, ones_v, stg_v, deg_sh,
                is0, is1, is2, is3, ss0, ss1, ss2, ss3):
    c = lax.axis_index("c")
    s = lax.axis_index("s")
    ibufs = (i0, i1, i2, i3)
    isems = (is0, is1, is2, is3)
    ssems = (ss0, ss1, ss2, ss3)
    one16 = jnp.full((16,), 1.0, dtype=jnp.float32)
    zero16 = jnp.zeros((16,), dtype=jnp.float32)
    for i in range(_CH // 16):
        ones_v[pl.ds(i * 16, 16)] = one16

    def fill0(i, _):
        stg_v[pl.ds(i * 16, 16)] = zero16
        return ()

    lax.fori_loop(0, 1008 // 16, fill0, ())

    @pl.when(s < 10)
    def _():
        pltpu.sync_copy(stg_v.at[pl.ds(0, 1000)],
                        deg_sh.at[pl.ds(s * 1000, 1000)])

    plsc.subcore_barrier()
    base = (c * _NS + s) * _EPT

    def i_start(ch, b):
        pltpu.async_copy(dst_hbm.at[pl.ds(base + ch * _CH, _CH)],
                         ibufs[b], isems[b])

    def i_wait(b):
        pltpu.make_async_copy(dst_hbm.at[pl.ds(0, _CH)],
                              ibufs[b], isems[b]).wait()

    def sc_start(b):
        pltpu.async_copy(ones_v, deg_sh.at[ibufs[b]], ssems[b], add=True)

    def sc_wait(b):
        pltpu.make_async_copy(ones_v, deg_sh.at[pl.ds(0, _CH)],
                              ssems[b]).wait()

    for b in range(4):
        i_start(b, b)

    def body(j, _):
        e = 4 * j
        for b in range(4):
            i_wait(b)
            sc_start(b)
        for b in range(4):
            sc_wait(b)

            @pl.when(e + 4 + b < _NCHUNK)
            def _():
                i_start(e + 4 + b, b)

        return ()

    lax.fori_loop(0, (_NCHUNK - 1) // 4, body, ())
    i_wait(0)
    pltpu.sync_copy(ones_v, deg_sh.at[i0], add=True)
    plsc.subcore_barrier()
    @pl.when(s < 10)
    def _():
        pltpu.sync_copy(deg_sh.at[pl.ds(s * 1000, 1000)],
                        stg_v.at[pl.ds(0, 1000)])
        pltpu.sync_copy(stg_v.at[pl.ds(0, 1000)],
                        deg_hbm.at[pl.ds(c * _N + s * 1000, 1000)])


@functools.partial(
    pl.kernel,
    out_type=jax.ShapeDtypeStruct((_NC * _N, _D), jnp.float32),
    mesh=_MESH,
    scratch_types=[
        pltpu.VMEM((_EPT,), jnp.int32),
        pltpu.VMEM((_CH,), jnp.int32),
        pltpu.VMEM((_CH,), jnp.int32),
        pltpu.VMEM((_CH, _D), jnp.float32),
        pltpu.VMEM((_CH, _D), jnp.float32),
        pltpu.VMEM_SHARED((_N, _D), jnp.float32),
        pltpu.SemaphoreType.DMA,
        pltpu.SemaphoreType.DMA,
        pltpu.SemaphoreType.DMA,
        pltpu.SemaphoreType.DMA,
        pltpu.SemaphoreType.DMA,
        pltpu.SemaphoreType.DMA,
    ],
)
def _edge_acc_kernel(y_hbm, src_hbm, dst_hbm, acc_hbm, sidx,
                     idxd0, idxd1, rows0, rows1, acc_sh,
                     gsem0, gsem1, ssem0, ssem1, isem0, isem1):
    c = lax.axis_index("c")
    s = lax.axis_index("s")
    base = (c * _NS + s) * _EPT
    pltpu.sync_copy(src_hbm.at[pl.ds(base, _EPT)], sidx)
    zero16 = jnp.zeros((16,), dtype=jnp.float32)

    def fill0(j, _):
        for i in range(_D // 16):
            rows0[j, pl.ds(i * 16, 16)] = zero16
        return ()

    lax.fori_loop(0, _CH, fill0, ())

    for k in range(8):
        @pl.when(s + 16 * k < _N // _CH)
        def _():
            pltpu.sync_copy(rows0,
                            acc_sh.at[pl.ds((s + 16 * k) * _CH, _CH), :])

    plsc.subcore_barrier()

    def g_start(ch, rbuf, sem):
        pltpu.async_copy(y_hbm.at[sidx.at[pl.ds(ch * _CH, _CH)]], rbuf, sem)

    def g_wait(rbuf, sem):
        pltpu.make_async_copy(y_hbm.at[pl.ds(0, _CH), :], rbuf, sem).wait()

    def sc_start(rbuf, ibuf, sem):
        pltpu.async_copy(rbuf, acc_sh.at[ibuf], sem, add=True)

    def sc_wait(rbuf, sem):
        pltpu.make_async_copy(rbuf, acc_sh.at[pl.ds(0, _CH), :], sem).wait()

    def i_start(ch, ibuf, sem):
        pltpu.async_copy(dst_hbm.at[pl.ds(base + ch * _CH, _CH)], ibuf, sem)

    def i_wait(ibuf, sem):
        pltpu.make_async_copy(dst_hbm.at[pl.ds(0, _CH)], ibuf, sem).wait()

    i_start(0, idxd0, isem0)
    i_start(1, idxd1, isem1)
    g_start(0, rows0, gsem0)
    g_start(1, rows1, gsem1)

    def body(j, _):
        e = 2 * j
        g_wait(rows0, gsem0)
        i_wait(idxd0, isem0)
        sc_start(rows0, idxd0, ssem0)
        g_wait(rows1, gsem1)
        i_wait(idxd1, isem1)
        sc_start(rows1, idxd1, ssem1)
        sc_wait(rows0, ssem0)
        i_start(e + 2, idxd0, isem0)
        g_start(e + 2, rows0, gsem0)
        sc_wait(rows1, ssem1)

        @pl.when(e + 3 < _NCHUNK)
        def _():
            i_start(e + 3, idxd1, isem1)
            g_start(e + 3, rows1, gsem1)

        return ()

    lax.fori_loop(0, (_NCHUNK - 1) // 2, body, ())
    g_wait(rows0, gsem0)
    i_wait(idxd0, isem0)
    pltpu.sync_copy(rows0, acc_sh.at[idxd0], add=True)
    plsc.subcore_barrier()
    for k in range(8):
        @pl.when(s + 16 * k < _N // _CH)
        def _():
            off = (s + 16 * k) * _CH
            pltpu.sync_copy(acc_sh.at[pl.ds(off, _CH), :], rows0)
            pltpu.sync_copy(rows0, acc_hbm.at[pl.ds(c * _N + off, _CH), :])


def _tc_mm_body(x_ref, w_ref, o_ref):
    o_ref[...] = jnp.dot(x_ref[...], w_ref[...],
                         preferred_element_type=jnp.float32)


def _tc_scale_body(degt_ref, xw_ref, dinv_ref, y_ref):
    deg = degt_ref[:, 0:1] + degt_ref[:, 1:2] + 1.0
    dinv = lax.rsqrt(deg)
    dinv_ref[...] = dinv
    y_ref[...] = xw_ref[...] * dinv


def _tc_mid_body(acc_ref, y_ref, dinv_ref, w_ref, b_ref, y2_ref):
    dinv = dinv_ref[...]
    h = dinv * (acc_ref[0:_N, :] + acc_ref[_N:, :] + y_ref[...]) + b_ref[...]
    hw = jnp.dot(h, w_ref[...], preferred_element_type=jnp.float32)
    y2_ref[...] = hw * dinv


def _tc_tail_body(acc_ref, y_ref, dinv_ref, b_ref, o_ref):
    dinv = dinv_ref[...]
    o_ref[...] = dinv * (acc_ref[0:_N, :] + acc_ref[_N:, :] + y_ref[...]) + b_ref[...]


def kernel(x, edge_index, W1, b1, W2, b2):
    src = edge_index[0]
    dst = edge_index[1]

    degp = _deg_kernel(dst)
    degt = jnp.transpose(degp.reshape(_NC, _N))

    xw1 = pl.pallas_call(
        _tc_mm_body,
        out_shape=jax.ShapeDtypeStruct((_N, _D), jnp.float32),
    )(x, W1)

    dinv, y1 = pl.pallas_call(
        _tc_scale_body,
        out_shape=(
            jax.ShapeDtypeStruct((_N, 1), jnp.float32),
            jax.ShapeDtypeStruct((_N, _D), jnp.float32),
        ),
    )(degt, xw1)

    acc1 = _edge_acc_kernel(y1, src, dst)

    y2 = pl.pallas_call(
        _tc_mid_body,
        out_shape=jax.ShapeDtypeStruct((_N, _D), jnp.float32),
    )(acc1, y1, dinv, W2, jnp.broadcast_to(b1, (_N, _D)))

    acc2 = _edge_acc_kernel(y2, src, dst)

    out = pl.pallas_call(
        _tc_tail_body,
        out_shape=jax.ShapeDtypeStruct((_N, _D), jnp.float32),
    )(acc2, y2, dinv, jnp.broadcast_to(b2, (_N, _D)))
    return out


# speedup vs baseline: 26.1284x; 1.0398x over previous
import functools

import jax
import jax.numpy as jnp
from jax import lax
from jax.experimental import pallas as pl
from jax.experimental.pallas import tpu as pltpu
from jax.experimental.pallas import tpu_sc as plsc

_N = 10000
_D = 128
_E = 320000
_NC = 2
_NS = 16
_NW = _NC * _NS
_EPT = _E // _NW
_CH = 80
_NCHUNK = _EPT // _CH
_CHP = 128
_EPTP = 10240
_PAD = _EPTP - _EPT
_NCHP = _EPTP // _CHP
_NPADROW = 256

_MESH = plsc.VectorSubcoreMesh(
    core_axis_name="c", subcore_axis_name="s", num_cores=_NC, num_subcores=_NS
)


@functools.partial(
    pl.kernel,
    out_type=jax.ShapeDtypeStruct((_NC * _N,), jnp.float32),
    mesh=_MESH,
    scratch_types=[
        pltpu.VMEM((_CH,), jnp.int32),
        pltpu.VMEM((_CH,), jnp.int32),
        pltpu.VMEM((_CH,), jnp.int32),
        pltpu.VMEM((_CH,), jnp.int32),
        pltpu.VMEM((_CH,), jnp.float32),
        pltpu.VMEM((1008,), jnp.float32),
        pltpu.VMEM_SHARED((_N,), jnp.float32),
        pltpu.SemaphoreType.DMA,
        pltpu.SemaphoreType.DMA,
        pltpu.SemaphoreType.DMA,
        pltpu.SemaphoreType.DMA,
        pltpu.SemaphoreType.DMA,
        pltpu.SemaphoreType.DMA,
        pltpu.SemaphoreType.DMA,
        pltpu.SemaphoreType.DMA,
    ],
)
def _deg_kernel(dst_hbm, deg_hbm, i0, i1, i2, i3, ones_v, stg_v, deg_sh,
                is0, is1, is2, is3, ss0, ss1, ss2, ss3):
    c = lax.axis_index("c")
    s = lax.axis_index("s")
    ibufs = (i0, i1, i2, i3)
    isems = (is0, is1, is2, is3)
    ssems = (ss0, ss1, ss2, ss3)
    one16 = jnp.full((16,), 1.0, dtype=jnp.float32)
    zero16 = jnp.zeros((16,), dtype=jnp.float32)
    for i in range(_CH // 16):
        ones_v[pl.ds(i * 16, 16)] = one16

    def fill0(i, _):
        stg_v[pl.ds(i * 16, 16)] = zero16
        return ()

    lax.fori_loop(0, 1008 // 16, fill0, ())

    @pl.when(s < 10)
    def _():
        pltpu.sync_copy(stg_v.at[pl.ds(0, 1000)],
                        deg_sh.at[pl.ds(s * 1000, 1000)])

    plsc.subcore_barrier()
    base = (c * _NS + s) * _EPT

    def i_start(ch, b):
        pltpu.async_copy(dst_hbm.at[pl.ds(base + ch * _CH, _CH)],
                         ibufs[b], isems[b])

    def i_wait(b):
        pltpu.make_async_copy(dst_hbm.at[pl.ds(0, _CH)],
                              ibufs[b], isems[b]).wait()

    def sc_start(b):
        pltpu.async_copy(ones_v, deg_sh.at[ibufs[b]], ssems[b], add=True)

    def sc_wait(b):
        pltpu.make_async_copy(ones_v, deg_sh.at[pl.ds(0, _CH)],
                              ssems[b]).wait()

    for b in range(4):
        i_start(b, b)

    def body(j, _):
        e = 4 * j
        for b in range(4):
            i_wait(b)
            sc_start(b)
        for b in range(4):
            sc_wait(b)

            @pl.when(e + 4 + b < _NCHUNK)
            def _():
                i_start(e + 4 + b, b)

        return ()

    lax.fori_loop(0, (_NCHUNK - 1) // 4, body, ())
    i_wait(0)
    pltpu.sync_copy(ones_v, deg_sh.at[i0], add=True)
    plsc.subcore_barrier()
    @pl.when(s < 10)
    def _():
        pltpu.sync_copy(deg_sh.at[pl.ds(s * 1000, 1000)],
                        stg_v.at[pl.ds(0, 1000)])
        pltpu.sync_copy(stg_v.at[pl.ds(0, 1000)],
                        deg_hbm.at[pl.ds(c * _N + s * 1000, 1000)])


@functools.partial(
    pl.kernel,
    out_type=jax.ShapeDtypeStruct((_NC * _N, _D), jnp.float32),
    mesh=_MESH,
    scratch_types=[
        pltpu.VMEM((_EPTP,), jnp.int32),
        pltpu.VMEM((_CHP,), jnp.int32),
        pltpu.VMEM((_CHP,), jnp.int32),
        pltpu.VMEM((_CHP, _D), jnp.float32),
        pltpu.VMEM((_CHP, _D), jnp.float32),
        pltpu.VMEM_SHARED((_N + _NPADROW, _D), jnp.float32),
        pltpu.SemaphoreType.DMA,
        pltpu.SemaphoreType.DMA,
        pltpu.SemaphoreType.DMA,
        pltpu.SemaphoreType.DMA,
        pltpu.SemaphoreType.DMA,
        pltpu.SemaphoreType.DMA,
    ],
)
def _edge_acc_kernel(y_hbm, src_hbm, dst_hbm, acc_hbm, sidx,
                     idxd0, idxd1, rows0, rows1, acc_sh,
                     gsem0, gsem1, ssem0, ssem1, isem0, isem1):
    c = lax.axis_index("c")
    s = lax.axis_index("s")
    base = (c * _NS + s) * _EPTP
    pltpu.sync_copy(src_hbm.at[pl.ds(base, _EPTP)], sidx)
    zero16 = jnp.zeros((16,), dtype=jnp.float32)

    def fill0(j, _):
        for i in range(_D // 16):
            rows0[j, pl.ds(i * 16, 16)] = zero16
        return ()

    lax.fori_loop(0, _CH, fill0, ())

    for k in range(8):
        @pl.when(s + 16 * k < _N // _CH)
        def _():
            pltpu.sync_copy(rows0.at[pl.ds(0, _CH), :],
                            acc_sh.at[pl.ds((s + 16 * k) * _CH, _CH), :])

    plsc.subcore_barrier()

    def g_start(ch, rbuf, sem):
        pltpu.async_copy(y_hbm.at[sidx.at[pl.ds(ch * _CHP, _CHP)]], rbuf, sem)

    def g_wait(rbuf, sem):
        pltpu.make_async_copy(y_hbm.at[pl.ds(0, _CHP), :], rbuf, sem).wait()

    def sc_start(rbuf, ibuf, sem):
        pltpu.async_copy(rbuf, acc_sh.at[ibuf], sem, add=True)

    def sc_wait(rbuf, sem):
        pltpu.make_async_copy(rbuf, acc_sh.at[pl.ds(0, _CHP), :], sem).wait()

    def i_start(ch, ibuf, sem):
        pltpu.async_copy(dst_hbm.at[pl.ds(base + ch * _CHP, _CHP)], ibuf, sem)

    def i_wait(ibuf, sem):
        pltpu.make_async_copy(dst_hbm.at[pl.ds(0, _CHP)], ibuf, sem).wait()

    i_start(0, idxd0, isem0)
    i_start(1, idxd1, isem1)
    g_start(0, rows0, gsem0)
    g_start(1, rows1, gsem1)

    def body(j, _):
        e = 2 * j
        g_wait(rows0, gsem0)
        i_wait(idxd0, isem0)
        sc_start(rows0, idxd0, ssem0)
        g_wait(rows1, gsem1)
        i_wait(idxd1, isem1)
        sc_start(rows1, idxd1, ssem1)
        sc_wait(rows0, ssem0)

        @pl.when(e + 2 < _NCHP)
        def _():
            i_start(e + 2, idxd0, isem0)
            g_start(e + 2, rows0, gsem0)

        sc_wait(rows1, ssem1)

        @pl.when(e + 3 < _NCHP)
        def _():
            i_start(e + 3, idxd1, isem1)
            g_start(e + 3, rows1, gsem1)

        return ()

    lax.fori_loop(0, _NCHP // 2, body, ())
    plsc.subcore_barrier()
    for k in range(8):
        @pl.when(s + 16 * k < _N // _CH)
        def _():
            off = (s + 16 * k) * _CH
            pltpu.sync_copy(acc_sh.at[pl.ds(off, _CH), :],
                            rows0.at[pl.ds(0, _CH), :])
            pltpu.sync_copy(rows0.at[pl.ds(0, _CH), :],
                            acc_hbm.at[pl.ds(c * _N + off, _CH), :])


def _tc_mm_body(x_ref, w_ref, o_ref):
    o_ref[...] = jnp.dot(x_ref[...], w_ref[...],
                         preferred_element_type=jnp.float32)


def _tc_scale_body(degt_ref, xw_ref, dinv_ref, y_ref):
    deg = degt_ref[:, 0:1] + degt_ref[:, 1:2] + 1.0
    dinv = lax.rsqrt(deg)
    dinv_ref[...] = dinv
    y_ref[...] = xw_ref[...] * dinv


def _tc_mid_body(acc_ref, y_ref, dinv_ref, w_ref, b_ref, y2_ref):
    dinv = dinv_ref[...]
    h = dinv * (acc_ref[0:_N, :] + acc_ref[_N:, :] + y_ref[...]) + b_ref[...]
    hw = jnp.dot(h, w_ref[...], preferred_element_type=jnp.float32)
    y2_ref[...] = hw * dinv


def _tc_tail_body(acc_ref, y_ref, dinv_ref, b_ref, o_ref):
    dinv = dinv_ref[...]
    o_ref[...] = dinv * (acc_ref[0:_N, :] + acc_ref[_N:, :] + y_ref[...]) + b_ref[...]


def kernel(x, edge_index, W1, b1, W2, b2):
    src = edge_index[0]
    dst = edge_index[1]

    iw = jnp.arange(_NW, dtype=jnp.int32)[:, None]
    ip = jnp.arange(_PAD, dtype=jnp.int32)[None, :]
    pad_s = (iw * 313 + ip * 41) % _N
    pad_d = _N + (iw * 8 + ip) % _NPADROW
    src_pad = jnp.concatenate([src.reshape(_NW, _EPT), pad_s], axis=1).reshape(-1)
    dst_pad = jnp.concatenate([dst.reshape(_NW, _EPT), pad_d], axis=1).reshape(-1)

    degp = _deg_kernel(dst)
    degt = jnp.transpose(degp.reshape(_NC, _N))

    xw1 = pl.pallas_call(
        _tc_mm_body,
        out_shape=jax.ShapeDtypeStruct((_N, _D), jnp.float32),
    )(x, W1)

    dinv, y1 = pl.pallas_call(
        _tc_scale_body,
        out_shape=(
            jax.ShapeDtypeStruct((_N, 1), jnp.float32),
            jax.ShapeDtypeStruct((_N, _D), jnp.float32),
        ),
    )(degt, xw1)

    acc1 = _edge_acc_kernel(y1, src_pad, dst_pad)

    y2 = pl.pallas_call(
        _tc_mid_body,
        out_shape=jax.ShapeDtypeStruct((_N, _D), jnp.float32),
    )(acc1, y1, dinv, W2, jnp.broadcast_to(b1, (_N, _D)))

    acc2 = _edge_acc_kernel(y2, src_pad, dst_pad)

    out = pl.pallas_call(
        _tc_tail_body,
        out_shape=jax.ShapeDtypeStruct((_N, _D), jnp.float32),
    )(acc2, y2, dinv, jnp.broadcast_to(b2, (_N, _D)))
    return out


# speedup vs baseline: 29.9436x; 1.1460x over previous
import functools

import jax
import jax.numpy as jnp
from jax import lax
from jax.experimental import pallas as pl
from jax.experimental.pallas import tpu as pltpu
from jax.experimental.pallas import tpu_sc as plsc

_N = 10000
_D = 128
_E = 320000
_NC = 2
_NS = 16
_NW = _NC * _NS
_EPT = _E // _NW
_CH = 80
_NCHUNK = _EPT // _CH
_CHP = 96
_EPTP = 10080
_PAD = _EPTP - _EPT
_NCHP = _EPTP // _CHP
_DEPTH = 3
_NPADROW = 256

_MESH = plsc.VectorSubcoreMesh(
    core_axis_name="c", subcore_axis_name="s", num_cores=_NC, num_subcores=_NS
)


@functools.partial(
    pl.kernel,
    out_type=jax.ShapeDtypeStruct((_NC * _N,), jnp.float32),
    mesh=_MESH,
    scratch_types=[
        pltpu.VMEM((_CH,), jnp.int32),
        pltpu.VMEM((_CH,), jnp.int32),
        pltpu.VMEM((_CH,), jnp.int32),
        pltpu.VMEM((_CH,), jnp.int32),
        pltpu.VMEM((_CH,), jnp.float32),
        pltpu.VMEM((1008,), jnp.float32),
        pltpu.VMEM_SHARED((_N,), jnp.float32),
        pltpu.SemaphoreType.DMA,
        pltpu.SemaphoreType.DMA,
        pltpu.SemaphoreType.DMA,
        pltpu.SemaphoreType.DMA,
        pltpu.SemaphoreType.DMA,
        pltpu.SemaphoreType.DMA,
        pltpu.SemaphoreType.DMA,
        pltpu.SemaphoreType.DMA,
    ],
)
def _deg_kernel(dst_hbm, deg_hbm, i0, i1, i2, i3, ones_v, stg_v, deg_sh,
                is0, is1, is2, is3, ss0, ss1, ss2, ss3):
    c = lax.axis_index("c")
    s = lax.axis_index("s")
    ibufs = (i0, i1, i2, i3)
    isems = (is0, is1, is2, is3)
    ssems = (ss0, ss1, ss2, ss3)
    one16 = jnp.full((16,), 1.0, dtype=jnp.float32)
    zero16 = jnp.zeros((16,), dtype=jnp.float32)
    for i in range(_CH // 16):
        ones_v[pl.ds(i * 16, 16)] = one16

    def fill0(i, _):
        stg_v[pl.ds(i * 16, 16)] = zero16
        return ()

    lax.fori_loop(0, 1008 // 16, fill0, ())

    @pl.when(s < 10)
    def _():
        pltpu.sync_copy(stg_v.at[pl.ds(0, 1000)],
                        deg_sh.at[pl.ds(s * 1000, 1000)])

    plsc.subcore_barrier()
    base = (c * _NS + s) * _EPT

    def i_start(ch, b):
        pltpu.async_copy(dst_hbm.at[pl.ds(base + ch * _CH, _CH)],
                         ibufs[b], isems[b])

    def i_wait(b):
        pltpu.make_async_copy(dst_hbm.at[pl.ds(0, _CH)],
                              ibufs[b], isems[b]).wait()

    def sc_start(b):
        pltpu.async_copy(ones_v, deg_sh.at[ibufs[b]], ssems[b], add=True)

    def sc_wait(b):
        pltpu.make_async_copy(ones_v, deg_sh.at[pl.ds(0, _CH)],
                              ssems[b]).wait()

    for b in range(4):
        i_start(b, b)

    def body(j, _):
        e = 4 * j
        for b in range(4):
            i_wait(b)
            sc_start(b)
        for b in range(4):
            sc_wait(b)

            @pl.when(e + 4 + b < _NCHUNK)
            def _():
                i_start(e + 4 + b, b)

        return ()

    lax.fori_loop(0, (_NCHUNK - 1) // 4, body, ())
    i_wait(0)
    pltpu.sync_copy(ones_v, deg_sh.at[i0], add=True)
    plsc.subcore_barrier()
    @pl.when(s < 10)
    def _():
        pltpu.sync_copy(deg_sh.at[pl.ds(s * 1000, 1000)],
                        stg_v.at[pl.ds(0, 1000)])
        pltpu.sync_copy(stg_v.at[pl.ds(0, 1000)],
                        deg_hbm.at[pl.ds(c * _N + s * 1000, 1000)])


@functools.partial(
    pl.kernel,
    out_type=jax.ShapeDtypeStruct((_NC * _N, _D), jnp.float32),
    mesh=_MESH,
    scratch_types=[
        pltpu.VMEM((_EPTP,), jnp.int32),
    ] + [pltpu.VMEM((_CHP,), jnp.int32) for _ in range(_DEPTH)]
      + [pltpu.VMEM((_CHP, _D), jnp.float32) for _ in range(_DEPTH)]
      + [pltpu.VMEM_SHARED((_N + _NPADROW, _D), jnp.float32)]
      + [pltpu.SemaphoreType.DMA for _ in range(3 * _DEPTH)],
)
def _edge_acc_kernel(y_hbm, src_hbm, dst_hbm, acc_hbm, sidx, *bufs):
    ibufs = bufs[:_DEPTH]
    rbufs = bufs[_DEPTH:2 * _DEPTH]
    acc_sh = bufs[2 * _DEPTH]
    gsems = bufs[2 * _DEPTH + 1:2 * _DEPTH + 1 + _DEPTH]
    ssems = bufs[2 * _DEPTH + 1 + _DEPTH:2 * _DEPTH + 1 + 2 * _DEPTH]
    isems = bufs[2 * _DEPTH + 1 + 2 * _DEPTH:]
    rows0 = rbufs[0]
    c = lax.axis_index("c")
    s = lax.axis_index("s")
    base = (c * _NS + s) * _EPTP
    pltpu.sync_copy(src_hbm.at[pl.ds(base, _EPTP)], sidx)
    zero16 = jnp.zeros((16,), dtype=jnp.float32)

    def fill0(j, _):
        for i in range(_D // 16):
            rows0[j, pl.ds(i * 16, 16)] = zero16
        return ()

    lax.fori_loop(0, _CH, fill0, ())

    for k in range(8):
        @pl.when(s + 16 * k < _N // _CH)
        def _():
            pltpu.sync_copy(rows0.at[pl.ds(0, _CH), :],
                            acc_sh.at[pl.ds((s + 16 * k) * _CH, _CH), :])

    plsc.subcore_barrier()

    def g_start(ch, b):
        pltpu.async_copy(y_hbm.at[sidx.at[pl.ds(ch * _CHP, _CHP)]],
                         rbufs[b], gsems[b])

    def g_wait(b):
        pltpu.make_async_copy(y_hbm.at[pl.ds(0, _CHP), :],
                              rbufs[b], gsems[b]).wait()

    def sc_start(b):
        pltpu.async_copy(rbufs[b], acc_sh.at[ibufs[b]], ssems[b], add=True)

    def sc_wait(b):
        pltpu.make_async_copy(rbufs[b], acc_sh.at[pl.ds(0, _CHP), :],
                              ssems[b]).wait()

    def i_start(ch, b):
        pltpu.async_copy(dst_hbm.at[pl.ds(base + ch * _CHP, _CHP)],
                         ibufs[b], isems[b])

    def i_wait(b):
        pltpu.make_async_copy(dst_hbm.at[pl.ds(0, _CHP)],
                              ibufs[b], isems[b]).wait()

    for b in range(_DEPTH):
        i_start(b, b)
        g_start(b, b)

    def body(j, _):
        e = _DEPTH * j
        for b in range(_DEPTH):
            g_wait(b)
            i_wait(b)
            sc_start(b)
        for b in range(_DEPTH):
            sc_wait(b)

            @pl.when(e + _DEPTH + b < _NCHP)
            def _():
                i_start(e + _DEPTH + b, b)
                g_start(e + _DEPTH + b, b)

        return ()

    lax.fori_loop(0, _NCHP // _DEPTH, body, ())
    plsc.subcore_barrier()
    for k in range(8):
        @pl.when(s + 16 * k < _N // _CH)
        def _():
            off = (s + 16 * k) * _CH
            pltpu.sync_copy(acc_sh.at[pl.ds(off, _CH), :],
                            rows0.at[pl.ds(0, _CH), :])
            pltpu.sync_copy(rows0.at[pl.ds(0, _CH), :],
                            acc_hbm.at[pl.ds(c * _N + off, _CH), :])


def _tc_mm_body(x_ref, w_ref, o_ref):
    o_ref[...] = jnp.dot(x_ref[...], w_ref[...],
                         preferred_element_type=jnp.float32)


def _tc_scale_body(degt_ref, xw_ref, dinv_ref, y_ref):
    deg = degt_ref[:, 0:1] + degt_ref[:, 1:2] + 1.0
    dinv = lax.rsqrt(deg)
    dinv_ref[...] = dinv
    y_ref[...] = xw_ref[...] * dinv


def _tc_mid_body(acc_ref, y_ref, dinv_ref, w_ref, b_ref, y2_ref):
    dinv = dinv_ref[...]
    h = dinv * (acc_ref[0:_N, :] + acc_ref[_N:, :] + y_ref[...]) + b_ref[...]
    hw = jnp.dot(h, w_ref[...], preferred_element_type=jnp.float32)
    y2_ref[...] = hw * dinv


def _tc_tail_body(acc_ref, y_ref, dinv_ref, b_ref, o_ref):
    dinv = dinv_ref[...]
    o_ref[...] = dinv * (acc_ref[0:_N, :] + acc_ref[_N:, :] + y_ref[...]) + b_ref[...]


def kernel(x, edge_index, W1, b1, W2, b2):
    src = edge_index[0]
    dst = edge_index[1]

    iw = jnp.arange(_NW, dtype=jnp.int32)[:, None]
    ip = jnp.arange(_PAD, dtype=jnp.int32)[None, :]
    pad_s = (iw * 313 + ip * 41) % _N
    pad_d = _N + (iw * 8 + ip) % _NPADROW
    src_pad = jnp.concatenate([src.reshape(_NW, _EPT), pad_s], axis=1).reshape(-1)
    dst_pad = jnp.concatenate([dst.reshape(_NW, _EPT), pad_d], axis=1).reshape(-1)

    degp = _deg_kernel(dst)
    degt = jnp.transpose(degp.reshape(_NC, _N))

    xw1 = pl.pallas_call(
        _tc_mm_body,
        out_shape=jax.ShapeDtypeStruct((_N, _D), jnp.float32),
    )(x, W1)

    dinv, y1 = pl.pallas_call(
        _tc_scale_body,
        out_shape=(
            jax.ShapeDtypeStruct((_N, 1), jnp.float32),
            jax.ShapeDtypeStruct((_N, _D), jnp.float32),
        ),
    )(degt, xw1)

    acc1 = _edge_acc_kernel(y1, src_pad, dst_pad)

    y2 = pl.pallas_call(
        _tc_mid_body,
        out_shape=jax.ShapeDtypeStruct((_N, _D), jnp.float32),
    )(acc1, y1, dinv, W2, jnp.broadcast_to(b1, (_N, _D)))

    acc2 = _edge_acc_kernel(y2, src_pad, dst_pad)

    out = pl.pallas_call(
        _tc_tail_body,
        out_shape=jax.ShapeDtypeStruct((_N, _D), jnp.float32),
    )(acc2, y2, dinv, jnp.broadcast_to(b2, (_N, _D)))
    return out


# speedup vs baseline: 31.6477x; 1.0569x over previous
import functools

import jax
import jax.numpy as jnp
from jax import lax
from jax.experimental import pallas as pl
from jax.experimental.pallas import tpu as pltpu
from jax.experimental.pallas import tpu_sc as plsc

_N = 10000
_D = 128
_E = 320000
_NC = 2
_NS = 16
_NW = _NC * _NS
_EPT = _E // _NW
_CH = 80
_NCHUNK = _EPT // _CH
_CHP = 72
_EPTP = 10080
_PAD = _EPTP - _EPT
_NCHP = _EPTP // _CHP
_DEPTH = 4
_NPADROW = 256

_MESH = plsc.VectorSubcoreMesh(
    core_axis_name="c", subcore_axis_name="s", num_cores=_NC, num_subcores=_NS
)


@functools.partial(
    pl.kernel,
    out_type=jax.ShapeDtypeStruct((_NC * _N,), jnp.float32),
    mesh=_MESH,
    scratch_types=[
        pltpu.VMEM((_CH,), jnp.int32),
        pltpu.VMEM((_CH,), jnp.int32),
        pltpu.VMEM((_CH,), jnp.int32),
        pltpu.VMEM((_CH,), jnp.int32),
        pltpu.VMEM((_CH,), jnp.float32),
        pltpu.VMEM((1008,), jnp.float32),
        pltpu.VMEM_SHARED((_N,), jnp.float32),
        pltpu.SemaphoreType.DMA,
        pltpu.SemaphoreType.DMA,
        pltpu.SemaphoreType.DMA,
        pltpu.SemaphoreType.DMA,
        pltpu.SemaphoreType.DMA,
        pltpu.SemaphoreType.DMA,
        pltpu.SemaphoreType.DMA,
        pltpu.SemaphoreType.DMA,
    ],
)
def _deg_kernel(dst_hbm, deg_hbm, i0, i1, i2, i3, ones_v, stg_v, deg_sh,
                is0, is1, is2, is3, ss0, ss1, ss2, ss3):
    c = lax.axis_index("c")
    s = lax.axis_index("s")
    ibufs = (i0, i1, i2, i3)
    isems = (is0, is1, is2, is3)
    ssems = (ss0, ss1, ss2, ss3)
    one16 = jnp.full((16,), 1.0, dtype=jnp.float32)
    zero16 = jnp.zeros((16,), dtype=jnp.float32)
    for i in range(_CH // 16):
        ones_v[pl.ds(i * 16, 16)] = one16

    def fill0(i, _):
        stg_v[pl.ds(i * 16, 16)] = zero16
        return ()

    lax.fori_loop(0, 1008 // 16, fill0, ())

    @pl.when(s < 10)
    def _():
        pltpu.sync_copy(stg_v.at[pl.ds(0, 1000)],
                        deg_sh.at[pl.ds(s * 1000, 1000)])

    plsc.subcore_barrier()
    base = (c * _NS + s) * _EPT

    def i_start(ch, b):
        pltpu.async_copy(dst_hbm.at[pl.ds(base + ch * _CH, _CH)],
                         ibufs[b], isems[b])

    def i_wait(b):
        pltpu.make_async_copy(dst_hbm.at[pl.ds(0, _CH)],
                              ibufs[b], isems[b]).wait()

    def sc_start(b):
        pltpu.async_copy(ones_v, deg_sh.at[ibufs[b]], ssems[b], add=True)

    def sc_wait(b):
        pltpu.make_async_copy(ones_v, deg_sh.at[pl.ds(0, _CH)],
                              ssems[b]).wait()

    for b in range(4):
        i_start(b, b)

    def body(j, _):
        e = 4 * j
        for b in range(4):
            i_wait(b)
            sc_start(b)
        for b in range(4):
            sc_wait(b)

            @pl.when(e + 4 + b < _NCHUNK)
            def _():
                i_start(e + 4 + b, b)

        return ()

    lax.fori_loop(0, (_NCHUNK - 1) // 4, body, ())
    i_wait(0)
    pltpu.sync_copy(ones_v, deg_sh.at[i0], add=True)
    plsc.subcore_barrier()
    @pl.when(s < 10)
    def _():
        pltpu.sync_copy(deg_sh.at[pl.ds(s * 1000, 1000)],
                        stg_v.at[pl.ds(0, 1000)])
        pltpu.sync_copy(stg_v.at[pl.ds(0, 1000)],
                        deg_hbm.at[pl.ds(c * _N + s * 1000, 1000)])


@functools.partial(
    pl.kernel,
    out_type=jax.ShapeDtypeStruct((_NC * _N, _D), jnp.float32),
    mesh=_MESH,
    scratch_types=[
        pltpu.VMEM((_EPTP,), jnp.int32),
    ] + [pltpu.VMEM((_CHP,), jnp.int32) for _ in range(_DEPTH)]
      + [pltpu.VMEM((_CHP, _D), jnp.float32) for _ in range(_DEPTH)]
      + [pltpu.VMEM_SHARED((_N + _NPADROW, _D), jnp.float32)]
      + [pltpu.SemaphoreType.DMA for _ in range(3 * _DEPTH)],
)
def _edge_acc_kernel(y_hbm, src_hbm, dst_hbm, acc_hbm, sidx, *bufs):
    ibufs = bufs[:_DEPTH]
    rbufs = bufs[_DEPTH:2 * _DEPTH]
    acc_sh = bufs[2 * _DEPTH]
    gsems = bufs[2 * _DEPTH + 1:2 * _DEPTH + 1 + _DEPTH]
    ssems = bufs[2 * _DEPTH + 1 + _DEPTH:2 * _DEPTH + 1 + 2 * _DEPTH]
    isems = bufs[2 * _DEPTH + 1 + 2 * _DEPTH:]
    rows0 = rbufs[0]
    c = lax.axis_index("c")
    s = lax.axis_index("s")
    base = (c * _NS + s) * _EPTP
    pltpu.sync_copy(src_hbm.at[pl.ds(base, _EPTP)], sidx)
    zero16 = jnp.zeros((16,), dtype=jnp.float32)

    def fill0(j, _):
        for i in range(_D // 16):
            rows0[j, pl.ds(i * 16, 16)] = zero16
        return ()

    lax.fori_loop(0, _CH, fill0, ())

    for k in range(8):
        @pl.when(s + 16 * k < _N // _CH)
        def _():
            pltpu.sync_copy(rows0.at[pl.ds(0, _CH), :],
                            acc_sh.at[pl.ds((s + 16 * k) * _CH, _CH), :])

    plsc.subcore_barrier()

    def g_start(ch, b):
        pltpu.async_copy(y_hbm.at[sidx.at[pl.ds(ch * _CHP, _CHP)]],
                         rbufs[b], gsems[b])

    def g_wait(b):
        pltpu.make_async_copy(y_hbm.at[pl.ds(0, _CHP), :],
                              rbufs[b], gsems[b]).wait()

    def sc_start(b):
        pltpu.async_copy(rbufs[b], acc_sh.at[ibufs[b]], ssems[b], add=True)

    def sc_wait(b):
        pltpu.make_async_copy(rbufs[b], acc_sh.at[pl.ds(0, _CHP), :],
                              ssems[b]).wait()

    def i_start(ch, b):
        pltpu.async_copy(dst_hbm.at[pl.ds(base + ch * _CHP, _CHP)],
                         ibufs[b], isems[b])

    def i_wait(b):
        pltpu.make_async_copy(dst_hbm.at[pl.ds(0, _CHP)],
                              ibufs[b], isems[b]).wait()

    for b in range(_DEPTH):
        i_start(b, b)
        g_start(b, b)

    def body(j, _):
        e = _DEPTH * j
        for b in range(_DEPTH):
            g_wait(b)
            i_wait(b)
            sc_start(b)
        for b in range(_DEPTH):
            sc_wait(b)

            @pl.when(e + _DEPTH + b < _NCHP)
            def _():
                i_start(e + _DEPTH + b, b)
                g_start(e + _DEPTH + b, b)

        return ()

    lax.fori_loop(0, _NCHP // _DEPTH, body, ())
    plsc.subcore_barrier()
    for k in range(8):
        @pl.when(s + 16 * k < _N // _CH)
        def _():
            off = (s + 16 * k) * _CH
            pltpu.sync_copy(acc_sh.at[pl.ds(off, _CH), :],
                            rows0.at[pl.ds(0, _CH), :])
            pltpu.sync_copy(rows0.at[pl.ds(0, _CH), :],
                            acc_hbm.at[pl.ds(c * _N + off, _CH), :])


def _tc_mm_body(x_ref, w_ref, o_ref):
    o_ref[...] = jnp.dot(x_ref[...], w_ref[...],
                         preferred_element_type=jnp.float32)


def _tc_scale_body(degt_ref, xw_ref, dinv_ref, y_ref):
    deg = degt_ref[:, 0:1] + degt_ref[:, 1:2] + 1.0
    dinv = lax.rsqrt(deg)
    dinv_ref[...] = dinv
    y_ref[...] = xw_ref[...] * dinv


def _tc_mid_body(acc_ref, y_ref, dinv_ref, w_ref, b_ref, y2_ref):
    dinv = dinv_ref[...]
    h = dinv * (acc_ref[0:_N, :] + acc_ref[_N:, :] + y_ref[...]) + b_ref[...]
    hw = jnp.dot(h, w_ref[...], preferred_element_type=jnp.float32)
    y2_ref[...] = hw * dinv


def _tc_tail_body(acc_ref, y_ref, dinv_ref, b_ref, o_ref):
    dinv = dinv_ref[...]
    o_ref[...] = dinv * (acc_ref[0:_N, :] + acc_ref[_N:, :] + y_ref[...]) + b_ref[...]


def kernel(x, edge_index, W1, b1, W2, b2):
    src = edge_index[0]
    dst = edge_index[1]

    iw = jnp.arange(_NW, dtype=jnp.int32)[:, None]
    ip = jnp.arange(_PAD, dtype=jnp.int32)[None, :]
    pad_s = (iw * 313 + ip * 41) % _N
    pad_d = _N + (iw * 8 + ip) % _NPADROW
    src_pad = jnp.concatenate([src.reshape(_NW, _EPT), pad_s], axis=1).reshape(-1)
    dst_pad = jnp.concatenate([dst.reshape(_NW, _EPT), pad_d], axis=1).reshape(-1)

    degp = _deg_kernel(dst)
    degt = jnp.transpose(degp.reshape(_NC, _N))

    xw1 = pl.pallas_call(
        _tc_mm_body,
        out_shape=jax.ShapeDtypeStruct((_N, _D), jnp.float32),
    )(x, W1)

    dinv, y1 = pl.pallas_call(
        _tc_scale_body,
        out_shape=(
            jax.ShapeDtypeStruct((_N, 1), jnp.float32),
            jax.ShapeDtypeStruct((_N, _D), jnp.float32),
        ),
    )(degt, xw1)

    acc1 = _edge_acc_kernel(y1, src_pad, dst_pad)

    y2 = pl.pallas_call(
        _tc_mid_body,
        out_shape=jax.ShapeDtypeStruct((_N, _D), jnp.float32),
    )(acc1, y1, dinv, W2, jnp.broadcast_to(b1, (_N, _D)))

    acc2 = _edge_acc_kernel(y2, src_pad, dst_pad)

    out = pl.pallas_call(
        _tc_tail_body,
        out_shape=jax.ShapeDtypeStruct((_N, _D), jnp.float32),
    )(acc2, y2, dinv, jnp.broadcast_to(b2, (_N, _D)))
    return out


# speedup vs baseline: 31.9290x; 1.0089x over previous
import functools

import jax
import jax.numpy as jnp
from jax import lax
from jax.experimental import pallas as pl
from jax.experimental.pallas import tpu as pltpu
from jax.experimental.pallas import tpu_sc as plsc

_N = 10000
_D = 128
_E = 320000
_NC = 2
_NS = 16
_NW = _NC * _NS
_EPT = _E // _NW
_CH = 80
_NCHUNK = _EPT // _CH
_CHP = 56
_EPTP = 10080
_PAD = _EPTP - _EPT
_NCHP = _EPTP // _CHP
_DEPTH = 5
_NPADROW = 256

_MESH = plsc.VectorSubcoreMesh(
    core_axis_name="c", subcore_axis_name="s", num_cores=_NC, num_subcores=_NS
)


@functools.partial(
    pl.kernel,
    out_type=jax.ShapeDtypeStruct((_NC * _N,), jnp.float32),
    mesh=_MESH,
    scratch_types=[
        pltpu.VMEM((_CH,), jnp.int32),
        pltpu.VMEM((_CH,), jnp.int32),
        pltpu.VMEM((_CH,), jnp.int32),
        pltpu.VMEM((_CH,), jnp.int32),
        pltpu.VMEM((_CH,), jnp.float32),
        pltpu.VMEM((1008,), jnp.float32),
        pltpu.VMEM_SHARED((_N,), jnp.float32),
        pltpu.SemaphoreType.DMA,
        pltpu.SemaphoreType.DMA,
        pltpu.SemaphoreType.DMA,
        pltpu.SemaphoreType.DMA,
        pltpu.SemaphoreType.DMA,
        pltpu.SemaphoreType.DMA,
        pltpu.SemaphoreType.DMA,
        pltpu.SemaphoreType.DMA,
    ],
)
def _deg_kernel(dst_hbm, deg_hbm, i0, i1, i2, i3, ones_v, stg_v, deg_sh,
                is0, is1, is2, is3, ss0, ss1, ss2, ss3):
    c = lax.axis_index("c")
    s = lax.axis_index("s")
    ibufs = (i0, i1, i2, i3)
    isems = (is0, is1, is2, is3)
    ssems = (ss0, ss1, ss2, ss3)
    one16 = jnp.full((16,), 1.0, dtype=jnp.float32)
    zero16 = jnp.zeros((16,), dtype=jnp.float32)
    for i in range(_CH // 16):
        ones_v[pl.ds(i * 16, 16)] = one16

    def fill0(i, _):
        stg_v[pl.ds(i * 16, 16)] = zero16
        return ()

    lax.fori_loop(0, 1008 // 16, fill0, ())

    @pl.when(s < 10)
    def _():
        pltpu.sync_copy(stg_v.at[pl.ds(0, 1000)],
                        deg_sh.at[pl.ds(s * 1000, 1000)])

    plsc.subcore_barrier()
    base = (c * _NS + s) * _EPT

    def i_start(ch, b):
        pltpu.async_copy(dst_hbm.at[pl.ds(base + ch * _CH, _CH)],
                         ibufs[b], isems[b])

    def i_wait(b):
        pltpu.make_async_copy(dst_hbm.at[pl.ds(0, _CH)],
                              ibufs[b], isems[b]).wait()

    def sc_start(b):
        pltpu.async_copy(ones_v, deg_sh.at[ibufs[b]], ssems[b], add=True)

    def sc_wait(b):
        pltpu.make_async_copy(ones_v, deg_sh.at[pl.ds(0, _CH)],
                              ssems[b]).wait()

    for b in range(4):
        i_start(b, b)

    def body(j, _):
        e = 4 * j
        for b in range(4):
            i_wait(b)
            sc_start(b)
        for b in range(4):
            sc_wait(b)

            @pl.when(e + 4 + b < _NCHUNK)
            def _():
                i_start(e + 4 + b, b)

        return ()

    lax.fori_loop(0, (_NCHUNK - 1) // 4, body, ())
    i_wait(0)
    pltpu.sync_copy(ones_v, deg_sh.at[i0], add=True)
    plsc.subcore_barrier()
    @pl.when(s < 10)
    def _():
        pltpu.sync_copy(deg_sh.at[pl.ds(s * 1000, 1000)],
                        stg_v.at[pl.ds(0, 1000)])
        pltpu.sync_copy(stg_v.at[pl.ds(0, 1000)],
                        deg_hbm.at[pl.ds(c * _N + s * 1000, 1000)])


@functools.partial(
    pl.kernel,
    out_type=jax.ShapeDtypeStruct((_NC * _N, _D), jnp.float32),
    mesh=_MESH,
    scratch_types=[
        pltpu.VMEM((_EPTP,), jnp.int32),
    ] + [pltpu.VMEM((_CHP,), jnp.int32) for _ in range(_DEPTH)]
      + [pltpu.VMEM((_CHP, _D), jnp.float32) for _ in range(_DEPTH)]
      + [pltpu.VMEM_SHARED((_N + _NPADROW, _D), jnp.float32)]
      + [pltpu.SemaphoreType.DMA for _ in range(3 * _DEPTH)],
)
def _edge_acc_kernel(y_hbm, src_hbm, dst_hbm, acc_hbm, sidx, *bufs):
    ibufs = bufs[:_DEPTH]
    rbufs = bufs[_DEPTH:2 * _DEPTH]
    acc_sh = bufs[2 * _DEPTH]
    gsems = bufs[2 * _DEPTH + 1:2 * _DEPTH + 1 + _DEPTH]
    ssems = bufs[2 * _DEPTH + 1 + _DEPTH:2 * _DEPTH + 1 + 2 * _DEPTH]
    isems = bufs[2 * _DEPTH + 1 + 2 * _DEPTH:]
    rows0 = rbufs[0]
    c = lax.axis_index("c")
    s = lax.axis_index("s")
    base = (c * _NS + s) * _EPTP
    pltpu.sync_copy(src_hbm.at[pl.ds(base, _EPTP)], sidx)
    zero16 = jnp.zeros((16,), dtype=jnp.float32)

    def fill0(j, _):
        for i in range(_D // 16):
            rows0[j, pl.ds(i * 16, 16)] = zero16
        return ()

    lax.fori_loop(0, _CH, fill0, ())

    for k in range(8):
        @pl.when(s + 16 * k < _N // _CH)
        def _():
            pltpu.sync_copy(rows0.at[pl.ds(0, _CH), :],
                            acc_sh.at[pl.ds((s + 16 * k) * _CH, _CH), :])

    plsc.subcore_barrier()

    def g_start(ch, b):
        pltpu.async_copy(y_hbm.at[sidx.at[pl.ds(ch * _CHP, _CHP)]],
                         rbufs[b], gsems[b])

    def g_wait(b):
        pltpu.make_async_copy(y_hbm.at[pl.ds(0, _CHP), :],
                              rbufs[b], gsems[b]).wait()

    def sc_start(b):
        pltpu.async_copy(rbufs[b], acc_sh.at[ibufs[b]], ssems[b], add=True)

    def sc_wait(b):
        pltpu.make_async_copy(rbufs[b], acc_sh.at[pl.ds(0, _CHP), :],
                              ssems[b]).wait()

    def i_start(ch, b):
        pltpu.async_copy(dst_hbm.at[pl.ds(base + ch * _CHP, _CHP)],
                         ibufs[b], isems[b])

    def i_wait(b):
        pltpu.make_async_copy(dst_hbm.at[pl.ds(0, _CHP)],
                              ibufs[b], isems[b]).wait()

    for b in range(_DEPTH):
        i_start(b, b)
        g_start(b, b)

    def body(j, _):
        e = _DEPTH * j
        for b in range(_DEPTH):
            g_wait(b)
            i_wait(b)
            sc_start(b)
        for b in range(_DEPTH):
            sc_wait(b)

            @pl.when(e + _DEPTH + b < _NCHP)
            def _():
                i_start(e + _DEPTH + b, b)
                g_start(e + _DEPTH + b, b)

        return ()

    lax.fori_loop(0, _NCHP // _DEPTH, body, ())
    plsc.subcore_barrier()
    for k in range(8):
        @pl.when(s + 16 * k < _N // _CH)
        def _():
            off = (s + 16 * k) * _CH
            pltpu.sync_copy(acc_sh.at[pl.ds(off, _CH), :],
                            rows0.at[pl.ds(0, _CH), :])
            pltpu.sync_copy(rows0.at[pl.ds(0, _CH), :],
                            acc_hbm.at[pl.ds(c * _N + off, _CH), :])


def _tc_mm_body(x_ref, w_ref, o_ref):
    o_ref[...] = jnp.dot(x_ref[...], w_ref[...],
                         preferred_element_type=jnp.float32)


def _tc_scale_body(degt_ref, xw_ref, dinv_ref, y_ref):
    deg = degt_ref[:, 0:1] + degt_ref[:, 1:2] + 1.0
    dinv = lax.rsqrt(deg)
    dinv_ref[...] = dinv
    y_ref[...] = xw_ref[...] * dinv


def _tc_mid_body(acc_ref, y_ref, dinv_ref, w_ref, b_ref, y2_ref):
    dinv = dinv_ref[...]
    h = dinv * (acc_ref[0:_N, :] + acc_ref[_N:, :] + y_ref[...]) + b_ref[...]
    hw = jnp.dot(h, w_ref[...], preferred_element_type=jnp.float32)
    y2_ref[...] = hw * dinv


def _tc_tail_body(acc_ref, y_ref, dinv_ref, b_ref, o_ref):
    dinv = dinv_ref[...]
    o_ref[...] = dinv * (acc_ref[0:_N, :] + acc_ref[_N:, :] + y_ref[...]) + b_ref[...]


def kernel(x, edge_index, W1, b1, W2, b2):
    src = edge_index[0]
    dst = edge_index[1]

    iw = jnp.arange(_NW, dtype=jnp.int32)[:, None]
    ip = jnp.arange(_PAD, dtype=jnp.int32)[None, :]
    pad_s = (iw * 313 + ip * 41) % _N
    pad_d = _N + (iw * 8 + ip) % _NPADROW
    src_pad = jnp.concatenate([src.reshape(_NW, _EPT), pad_s], axis=1).reshape(-1)
    dst_pad = jnp.concatenate([dst.reshape(_NW, _EPT), pad_d], axis=1).reshape(-1)

    degp = _deg_kernel(dst)
    degt = jnp.transpose(degp.reshape(_NC, _N))

    xw1 = pl.pallas_call(
        _tc_mm_body,
        out_shape=jax.ShapeDtypeStruct((_N, _D), jnp.float32),
    )(x, W1)

    dinv, y1 = pl.pallas_call(
        _tc_scale_body,
        out_shape=(
            jax.ShapeDtypeStruct((_N, 1), jnp.float32),
            jax.ShapeDtypeStruct((_N, _D), jnp.float32),
        ),
    )(degt, xw1)

    acc1 = _edge_acc_kernel(y1, src_pad, dst_pad)

    y2 = pl.pallas_call(
        _tc_mid_body,
        out_shape=jax.ShapeDtypeStruct((_N, _D), jnp.float32),
    )(acc1, y1, dinv, W2, jnp.broadcast_to(b1, (_N, _D)))

    acc2 = _edge_acc_kernel(y2, src_pad, dst_pad)

    out = pl.pallas_call(
        _tc_tail_body,
        out_shape=jax.ShapeDtypeStruct((_N, _D), jnp.float32),
    )(acc2, y2, dinv, jnp.broadcast_to(b2, (_N, _D)))
    return out


# speedup vs baseline: 32.7519x; 1.0258x over previous
import functools

import jax
import jax.numpy as jnp
from jax import lax
from jax.experimental import pallas as pl
from jax.experimental.pallas import tpu as pltpu
from jax.experimental.pallas import tpu_sc as plsc

_N = 10000
_D = 128
_E = 320000
_NC = 2
_NS = 16
_NW = _NC * _NS
_EPT = _E // _NW
_CH = 80
_NCHUNK = _EPT // _CH
_CHP = 56
_EPTP = 10080
_PAD = _EPTP - _EPT
_NCHP = _EPTP // _CHP
_DEPTH = 5
_NPADROW = 256

_MESH = plsc.VectorSubcoreMesh(
    core_axis_name="c", subcore_axis_name="s", num_cores=_NC, num_subcores=_NS
)


@functools.partial(
    pl.kernel,
    out_type=jax.ShapeDtypeStruct((_NC * _N,), jnp.float32),
    mesh=_MESH,
    scratch_types=[
        pltpu.VMEM((_CH,), jnp.int32),
        pltpu.VMEM((_CH,), jnp.int32),
        pltpu.VMEM((_CH,), jnp.int32),
        pltpu.VMEM((_CH,), jnp.int32),
        pltpu.VMEM((_CH,), jnp.float32),
        pltpu.VMEM((1008,), jnp.float32),
        pltpu.VMEM_SHARED((_N,), jnp.float32),
        pltpu.SemaphoreType.DMA,
        pltpu.SemaphoreType.DMA,
        pltpu.SemaphoreType.DMA,
        pltpu.SemaphoreType.DMA,
        pltpu.SemaphoreType.DMA,
        pltpu.SemaphoreType.DMA,
        pltpu.SemaphoreType.DMA,
        pltpu.SemaphoreType.DMA,
    ],
)
def _deg_kernel(dst_hbm, deg_hbm, i0, i1, i2, i3, ones_v, stg_v, deg_sh,
                is0, is1, is2, is3, ss0, ss1, ss2, ss3):
    c = lax.axis_index("c")
    s = lax.axis_index("s")
    ibufs = (i0, i1, i2, i3)
    isems = (is0, is1, is2, is3)
    ssems = (ss0, ss1, ss2, ss3)
    one16 = jnp.full((16,), 1.0, dtype=jnp.float32)
    zero16 = jnp.zeros((16,), dtype=jnp.float32)
    for i in range(_CH // 16):
        ones_v[pl.ds(i * 16, 16)] = one16

    def fill0(i, _):
        stg_v[pl.ds(i * 16, 16)] = zero16
        return ()

    lax.fori_loop(0, 1008 // 16, fill0, ())

    @pl.when(s < 10)
    def _():
        pltpu.sync_copy(stg_v.at[pl.ds(0, 1000)],
                        deg_sh.at[pl.ds(s * 1000, 1000)])

    plsc.subcore_barrier()
    base = (c * _NS + s) * _EPT

    def i_start(ch, b):
        pltpu.async_copy(dst_hbm.at[pl.ds(base + ch * _CH, _CH)],
                         ibufs[b], isems[b])

    def i_wait(b):
        pltpu.make_async_copy(dst_hbm.at[pl.ds(0, _CH)],
                              ibufs[b], isems[b]).wait()

    def sc_start(b):
        pltpu.async_copy(ones_v, deg_sh.at[ibufs[b]], ssems[b], add=True)

    def sc_wait(b):
        pltpu.make_async_copy(ones_v, deg_sh.at[pl.ds(0, _CH)],
                              ssems[b]).wait()

    for b in range(4):
        i_start(b, b)

    def body(j, _):
        e = 4 * j
        for b in range(4):
            i_wait(b)
            sc_start(b)
        for b in range(4):
            sc_wait(b)

            @pl.when(e + 4 + b < _NCHUNK)
            def _():
                i_start(e + 4 + b, b)

        return ()

    lax.fori_loop(0, (_NCHUNK - 1) // 4, body, ())
    i_wait(0)
    pltpu.sync_copy(ones_v, deg_sh.at[i0], add=True)
    plsc.subcore_barrier()
    @pl.when(s < 10)
    def _():
        pltpu.sync_copy(deg_sh.at[pl.ds(s * 1000, 1000)],
                        stg_v.at[pl.ds(0, 1000)])
        pltpu.sync_copy(stg_v.at[pl.ds(0, 1000)],
                        deg_hbm.at[pl.ds(c * _N + s * 1000, 1000)])


@functools.partial(
    pl.kernel,
    out_type=jax.ShapeDtypeStruct((_NC * _N, _D), jnp.float32),
    mesh=_MESH,
    scratch_types=[
        pltpu.VMEM((_EPTP,), jnp.int32),
    ] + [pltpu.VMEM((_CHP,), jnp.int32) for _ in range(_DEPTH)]
      + [pltpu.VMEM((_CHP, _D), jnp.float32) for _ in range(_DEPTH)]
      + [pltpu.VMEM_SHARED((_N + _NPADROW, _D), jnp.float32)]
      + [pltpu.SemaphoreType.DMA for _ in range(3 * _DEPTH)],
)
def _edge_acc_kernel(y_hbm, src_hbm, dst_hbm, acc_hbm, sidx, *bufs):
    ibufs = bufs[:_DEPTH]
    rbufs = bufs[_DEPTH:2 * _DEPTH]
    acc_sh = bufs[2 * _DEPTH]
    gsems = bufs[2 * _DEPTH + 1:2 * _DEPTH + 1 + _DEPTH]
    ssems = bufs[2 * _DEPTH + 1 + _DEPTH:2 * _DEPTH + 1 + 2 * _DEPTH]
    isems = bufs[2 * _DEPTH + 1 + 2 * _DEPTH:]
    rows0 = rbufs[0]
    c = lax.axis_index("c")
    s = lax.axis_index("s")
    base = (c * _NS + s) * _EPTP
    pltpu.sync_copy(src_hbm.at[pl.ds(base, _EPTP)], sidx)
    zero16 = jnp.zeros((16,), dtype=jnp.float32)

    def fill0(j, _):
        for i in range(_D // 16):
            rows0[j, pl.ds(i * 16, 16)] = zero16
        return ()

    lax.fori_loop(0, _CH, fill0, ())

    for k in range(8):
        @pl.when(s + 16 * k < _N // _CH)
        def _():
            pltpu.sync_copy(rows0.at[pl.ds(0, _CH), :],
                            acc_sh.at[pl.ds((s + 16 * k) * _CH, _CH), :])

    plsc.subcore_barrier()

    def g_start(ch, b):
        pltpu.async_copy(y_hbm.at[sidx.at[pl.ds(ch * _CHP, _CHP)]],
                         rbufs[b], gsems[b])

    def g_wait(b):
        pltpu.make_async_copy(y_hbm.at[pl.ds(0, _CHP), :],
                              rbufs[b], gsems[b]).wait()

    def sc_start(b):
        pltpu.async_copy(rbufs[b], acc_sh.at[ibufs[b]], ssems[b], add=True)

    def sc_wait(b):
        pltpu.make_async_copy(rbufs[b], acc_sh.at[pl.ds(0, _CHP), :],
                              ssems[b]).wait()

    def i_start(ch, b):
        pltpu.async_copy(dst_hbm.at[pl.ds(base + ch * _CHP, _CHP)],
                         ibufs[b], isems[b])

    def i_wait(b):
        pltpu.make_async_copy(dst_hbm.at[pl.ds(0, _CHP)],
                              ibufs[b], isems[b]).wait()

    for b in range(_DEPTH):
        i_start(b, b)
        g_start(b, b)

    def body(j, _):
        e = _DEPTH * j
        for b in range(_DEPTH):
            g_wait(b)
            i_wait(b)
            sc_start(b)
        for b in range(_DEPTH):
            sc_wait(b)

            @pl.when(e + _DEPTH + b < _NCHP)
            def _():
                i_start(e + _DEPTH + b, b)
                g_start(e + _DEPTH + b, b)

        return ()

    lax.fori_loop(0, _NCHP // _DEPTH, body, ())
    plsc.subcore_barrier()
    for k in range(8):
        @pl.when(s + 16 * k < _N // _CH)
        def _():
            off = (s + 16 * k) * _CH
            pltpu.sync_copy(acc_sh.at[pl.ds(off, _CH), :],
                            rows0.at[pl.ds(0, _CH), :])
            pltpu.sync_copy(rows0.at[pl.ds(0, _CH), :],
                            acc_hbm.at[pl.ds(c * _N + off, _CH), :])


def _tc_mm_body(x_ref, w_ref, o_ref):
    o_ref[...] = jnp.dot(x_ref[...], w_ref[...],
                         preferred_element_type=jnp.float32)


def _tc_scale_body(degt_ref, xw_ref, dinv_ref, y_ref):
    deg = degt_ref[:, 0:1] + degt_ref[:, 1:2] + 1.0
    dinv = lax.rsqrt(deg)
    dinv_ref[...] = dinv
    y_ref[...] = xw_ref[...] * dinv


def _tc_mid_body(acc_ref, y_ref, dinv_ref, w_ref, b_ref, y2_ref):
    dinv = dinv_ref[...]
    h = dinv * (acc_ref[0:_N, :] + acc_ref[_N:, :] + y_ref[...]) + b_ref[...]
    hw = jnp.dot(h, w_ref[...], preferred_element_type=jnp.float32)
    y2_ref[...] = hw * dinv


def _tc_tail_body(acc_ref, y_ref, dinv_ref, b_ref, o_ref):
    dinv = dinv_ref[...]
    o_ref[...] = dinv * (acc_ref[0:_N, :] + acc_ref[_N:, :] + y_ref[...]) + b_ref[...]


def kernel(x, edge_index, W1, b1, W2, b2):
    src = edge_index[0]
    dst = edge_index[1]

    iw = jnp.arange(_NW, dtype=jnp.int32)[:, None]
    ip = jnp.arange(_PAD, dtype=jnp.int32)[None, :]
    pad_s = (iw * 313 + ip * 41) % _N
    pad_d = _N + (iw * 8 + ip) % _NPADROW
    src_pad = jnp.concatenate([src.reshape(_NW, _EPT), pad_s], axis=1).reshape(-1)
    dst_pad = jnp.concatenate([dst.reshape(_NW, _EPT), pad_d], axis=1).reshape(-1)

    degp = _deg_kernel(dst)
    degt = jnp.transpose(degp.reshape(_NC, _N))

    xw1 = pl.pallas_call(
        _tc_mm_body,
        out_shape=jax.ShapeDtypeStruct((_N, _D), jnp.float32),
    )(x, W1)

    dinv, y1 = pl.pallas_call(
        _tc_scale_body,
        out_shape=(
            jax.ShapeDtypeStruct((_N, 1), jnp.float32),
            jax.ShapeDtypeStruct((_N, _D), jnp.float32),
        ),
    )(degt, xw1)

    acc1 = _edge_acc_kernel(y1, src_pad, dst_pad)

    y2 = pl.pallas_call(
        _tc_mid_body,
        out_shape=jax.ShapeDtypeStruct((_N, _D), jnp.float32),
    )(acc1, y1, dinv, W2, b1.reshape(1, _D))

    acc2 = _edge_acc_kernel(y2, src_pad, dst_pad)

    out = pl.pallas_call(
        _tc_tail_body,
        out_shape=jax.ShapeDtypeStruct((_N, _D), jnp.float32),
    )(acc2, y2, dinv, b2.reshape(1, _D))
    return out


# speedup vs baseline: 33.0521x; 1.0092x over previous
import functools

import jax
import jax.numpy as jnp
from jax import lax
from jax.experimental import pallas as pl
from jax.experimental.pallas import tpu as pltpu
from jax.experimental.pallas import tpu_sc as plsc

_N = 10000
_D = 128
_E = 320000
_NC = 2
_NS = 16
_NW = _NC * _NS
_EPT = _E // _NW
_CH = 80
_CHD = 112
_NCHD = 10080 // _CHD
_CHP = 56
_EPTP = 10080
_PAD = _EPTP - _EPT
_NCHP = _EPTP // _CHP
_DEPTH = 5
_NPADROW = 256

_MESH = plsc.VectorSubcoreMesh(
    core_axis_name="c", subcore_axis_name="s", num_cores=_NC, num_subcores=_NS
)


@functools.partial(
    pl.kernel,
    out_type=jax.ShapeDtypeStruct((_NC * _N,), jnp.float32),
    mesh=_MESH,
    scratch_types=[
        pltpu.VMEM((_CHD,), jnp.int32),
        pltpu.VMEM((_CHD,), jnp.int32),
        pltpu.VMEM((_CHD,), jnp.int32),
        pltpu.VMEM((_CHD,), jnp.int32),
        pltpu.VMEM((_CHD,), jnp.float32),
        pltpu.VMEM((1008,), jnp.float32),
        pltpu.VMEM_SHARED((_N + _NPADROW,), jnp.float32),
        pltpu.SemaphoreType.DMA,
        pltpu.SemaphoreType.DMA,
        pltpu.SemaphoreType.DMA,
        pltpu.SemaphoreType.DMA,
        pltpu.SemaphoreType.DMA,
        pltpu.SemaphoreType.DMA,
        pltpu.SemaphoreType.DMA,
        pltpu.SemaphoreType.DMA,
    ],
)
def _deg_kernel(dst_hbm, deg_hbm, i0, i1, i2, i3, ones_v, stg_v, deg_sh,
                is0, is1, is2, is3, ss0, ss1, ss2, ss3):
    c = lax.axis_index("c")
    s = lax.axis_index("s")
    ibufs = (i0, i1, i2, i3)
    isems = (is0, is1, is2, is3)
    ssems = (ss0, ss1, ss2, ss3)
    one16 = jnp.full((16,), 1.0, dtype=jnp.float32)
    zero16 = jnp.zeros((16,), dtype=jnp.float32)
    for i in range(_CHD // 16):
        ones_v[pl.ds(i * 16, 16)] = one16

    def fill0(i, _):
        stg_v[pl.ds(i * 16, 16)] = zero16
        return ()

    lax.fori_loop(0, 1008 // 16, fill0, ())

    @pl.when(s < 10)
    def _():
        pltpu.sync_copy(stg_v.at[pl.ds(0, 1000)],
                        deg_sh.at[pl.ds(s * 1000, 1000)])

    @pl.when(s == 10)
    def _():
        pltpu.sync_copy(stg_v.at[pl.ds(0, _NPADROW)],
                        deg_sh.at[pl.ds(_N, _NPADROW)])

    plsc.subcore_barrier()
    base = (c * _NS + s) * _EPTP

    def i_start(ch, b):
        pltpu.async_copy(dst_hbm.at[pl.ds(base + ch * _CHD, _CHD)],
                         ibufs[b], isems[b])

    def i_wait(b):
        pltpu.make_async_copy(dst_hbm.at[pl.ds(0, _CHD)],
                              ibufs[b], isems[b]).wait()

    def sc_start(b):
        pltpu.async_copy(ones_v, deg_sh.at[ibufs[b]], ssems[b], add=True)

    def sc_wait(b):
        pltpu.make_async_copy(ones_v, deg_sh.at[pl.ds(0, _CHD)],
                              ssems[b]).wait()

    for b in range(4):
        i_start(b, b)

    def body(j, _):
        e = 4 * j
        for b in range(4):
            i_wait(b)
            sc_start(b)
        for b in range(4):
            sc_wait(b)

            @pl.when(e + 4 + b < _NCHD)
            def _():
                i_start(e + 4 + b, b)

        return ()

    lax.fori_loop(0, _NCHD // 4, body, ())
    for b in range(_NCHD % 4):
        i_wait(b)
        sc_start(b)
    for b in range(_NCHD % 4):
        sc_wait(b)
    plsc.subcore_barrier()
    @pl.when(s < 10)
    def _():
        pltpu.sync_copy(deg_sh.at[pl.ds(s * 1000, 1000)],
                        stg_v.at[pl.ds(0, 1000)])
        pltpu.sync_copy(stg_v.at[pl.ds(0, 1000)],
                        deg_hbm.at[pl.ds(c * _N + s * 1000, 1000)])


@functools.partial(
    pl.kernel,
    out_type=jax.ShapeDtypeStruct((_NC * _N, _D), jnp.float32),
    mesh=_MESH,
    scratch_types=[
        pltpu.VMEM((_EPTP,), jnp.int32),
    ] + [pltpu.VMEM((_CHP,), jnp.int32) for _ in range(_DEPTH)]
      + [pltpu.VMEM((_CHP, _D), jnp.float32) for _ in range(_DEPTH)]
      + [pltpu.VMEM_SHARED((_N + _NPADROW, _D), jnp.float32)]
      + [pltpu.SemaphoreType.DMA for _ in range(3 * _DEPTH)],
)
def _edge_acc_kernel(y_hbm, src_hbm, dst_hbm, acc_hbm, sidx, *bufs):
    ibufs = bufs[:_DEPTH]
    rbufs = bufs[_DEPTH:2 * _DEPTH]
    acc_sh = bufs[2 * _DEPTH]
    gsems = bufs[2 * _DEPTH + 1:2 * _DEPTH + 1 + _DEPTH]
    ssems = bufs[2 * _DEPTH + 1 + _DEPTH:2 * _DEPTH + 1 + 2 * _DEPTH]
    isems = bufs[2 * _DEPTH + 1 + 2 * _DEPTH:]
    rows0 = rbufs[0]
    c = lax.axis_index("c")
    s = lax.axis_index("s")
    base = (c * _NS + s) * _EPTP
    pltpu.sync_copy(src_hbm.at[pl.ds(base, _EPTP)], sidx)
    zero16 = jnp.zeros((16,), dtype=jnp.float32)

    def fill0(j, _):
        for i in range(_D // 16):
            rows0[j, pl.ds(i * 16, 16)] = zero16
        return ()

    lax.fori_loop(0, _CH, fill0, ())

    for k in range(8):
        @pl.when(s + 16 * k < _N // _CH)
        def _():
            pltpu.sync_copy(rows0.at[pl.ds(0, _CH), :],
                            acc_sh.at[pl.ds((s + 16 * k) * _CH, _CH), :])

    plsc.subcore_barrier()

    def g_start(ch, b):
        pltpu.async_copy(y_hbm.at[sidx.at[pl.ds(ch * _CHP, _CHP)]],
                         rbufs[b], gsems[b])

    def g_wait(b):
        pltpu.make_async_copy(y_hbm.at[pl.ds(0, _CHP), :],
                              rbufs[b], gsems[b]).wait()

    def sc_start(b):
        pltpu.async_copy(rbufs[b], acc_sh.at[ibufs[b]], ssems[b], add=True)

    def sc_wait(b):
        pltpu.make_async_copy(rbufs[b], acc_sh.at[pl.ds(0, _CHP), :],
                              ssems[b]).wait()

    def i_start(ch, b):
        pltpu.async_copy(dst_hbm.at[pl.ds(base + ch * _CHP, _CHP)],
                         ibufs[b], isems[b])

    def i_wait(b):
        pltpu.make_async_copy(dst_hbm.at[pl.ds(0, _CHP)],
                              ibufs[b], isems[b]).wait()

    for b in range(_DEPTH):
        i_start(b, b)
        g_start(b, b)

    def body(j, _):
        e = _DEPTH * j
        for b in range(_DEPTH):
            g_wait(b)
            i_wait(b)
            sc_start(b)
        for b in range(_DEPTH):
            sc_wait(b)

            @pl.when(e + _DEPTH + b < _NCHP)
            def _():
                i_start(e + _DEPTH + b, b)
                g_start(e + _DEPTH + b, b)

        return ()

    lax.fori_loop(0, _NCHP // _DEPTH, body, ())
    plsc.subcore_barrier()
    for k in range(8):
        @pl.when(s + 16 * k < _N // _CH)
        def _():
            off = (s + 16 * k) * _CH
            pltpu.sync_copy(acc_sh.at[pl.ds(off, _CH), :],
                            rows0.at[pl.ds(0, _CH), :])
            pltpu.sync_copy(rows0.at[pl.ds(0, _CH), :],
                            acc_hbm.at[pl.ds(c * _N + off, _CH), :])


def _tc_mm_body(x_ref, w_ref, o_ref):
    o_ref[...] = jnp.dot(x_ref[...], w_ref[...],
                         preferred_element_type=jnp.float32)


def _tc_scale_body(degt_ref, xw_ref, dinv_ref, y_ref):
    deg = degt_ref[:, 0:1] + degt_ref[:, 1:2] + 1.0
    dinv = lax.rsqrt(deg)
    dinv_ref[...] = dinv
    y_ref[...] = xw_ref[...] * dinv


def _tc_mid_body(acc_ref, y_ref, dinv_ref, w_ref, b_ref, y2_ref):
    dinv = dinv_ref[...]
    h = dinv * (acc_ref[0:_N, :] + acc_ref[_N:, :] + y_ref[...]) + b_ref[...]
    hw = jnp.dot(h, w_ref[...], preferred_element_type=jnp.float32)
    y2_ref[...] = hw * dinv


def _tc_tail_body(acc_ref, y_ref, dinv_ref, b_ref, o_ref):
    dinv = dinv_ref[...]
    o_ref[...] = dinv * (acc_ref[0:_N, :] + acc_ref[_N:, :] + y_ref[...]) + b_ref[...]


def kernel(x, edge_index, W1, b1, W2, b2):
    src = edge_index[0]
    dst = edge_index[1]

    iw = jnp.arange(_NW, dtype=jnp.int32)[:, None]
    ip = jnp.arange(_PAD, dtype=jnp.int32)[None, :]
    pad_s = (iw * 313 + ip * 41) % _N
    pad_d = _N + (iw * 8 + ip) % _NPADROW
    src_pad = jnp.concatenate([src.reshape(_NW, _EPT), pad_s], axis=1).reshape(-1)
    dst_pad = jnp.concatenate([dst.reshape(_NW, _EPT), pad_d], axis=1).reshape(-1)

    degp = _deg_kernel(dst_pad)
    degt = jnp.transpose(degp.reshape(_NC, _N))

    xw1 = pl.pallas_call(
        _tc_mm_body,
        out_shape=jax.ShapeDtypeStruct((_N, _D), jnp.float32),
    )(x, W1)

    dinv, y1 = pl.pallas_call(
        _tc_scale_body,
        out_shape=(
            jax.ShapeDtypeStruct((_N, 1), jnp.float32),
            jax.ShapeDtypeStruct((_N, _D), jnp.float32),
        ),
    )(degt, xw1)

    acc1 = _edge_acc_kernel(y1, src_pad, dst_pad)

    y2 = pl.pallas_call(
        _tc_mid_body,
        out_shape=jax.ShapeDtypeStruct((_N, _D), jnp.float32),
    )(acc1, y1, dinv, W2, b1.reshape(1, _D))

    acc2 = _edge_acc_kernel(y2, src_pad, dst_pad)

    out = pl.pallas_call(
        _tc_tail_body,
        out_shape=jax.ShapeDtypeStruct((_N, _D), jnp.float32),
    )(acc2, y2, dinv, b2.reshape(1, _D))
    return out


# speedup vs baseline: 33.4115x; 1.0109x over previous
import functools

import jax
import jax.numpy as jnp
from jax import lax
from jax.experimental import pallas as pl
from jax.experimental.pallas import tpu as pltpu
from jax.experimental.pallas import tpu_sc as plsc

_N = 10000
_D = 128
_E = 320000
_NC = 2
_NS = 16
_NW = _NC * _NS
_EPT = _E // _NW
_CHW = 40
_NCHW = _N // _CHW
_CHD = 112
_NCHD = 10080 // _CHD
_CHP = 56
_EPTP = 10080
_PAD = _EPTP - _EPT
_NCHP = _EPTP // _CHP
_DEPTH = 5
_NPADROW = 256

_MESH = plsc.VectorSubcoreMesh(
    core_axis_name="c", subcore_axis_name="s", num_cores=_NC, num_subcores=_NS
)


@functools.partial(
    pl.kernel,
    out_type=jax.ShapeDtypeStruct((_NC * _N,), jnp.float32),
    mesh=_MESH,
    scratch_types=[
        pltpu.VMEM((_CHD,), jnp.int32),
        pltpu.VMEM((_CHD,), jnp.int32),
        pltpu.VMEM((_CHD,), jnp.int32),
        pltpu.VMEM((_CHD,), jnp.int32),
        pltpu.VMEM((_CHD,), jnp.float32),
        pltpu.VMEM((1008,), jnp.float32),
        pltpu.VMEM_SHARED((_N + _NPADROW,), jnp.float32),
        pltpu.SemaphoreType.DMA,
        pltpu.SemaphoreType.DMA,
        pltpu.SemaphoreType.DMA,
        pltpu.SemaphoreType.DMA,
        pltpu.SemaphoreType.DMA,
        pltpu.SemaphoreType.DMA,
        pltpu.SemaphoreType.DMA,
        pltpu.SemaphoreType.DMA,
    ],
)
def _deg_kernel(dst_hbm, deg_hbm, i0, i1, i2, i3, ones_v, stg_v, deg_sh,
                is0, is1, is2, is3, ss0, ss1, ss2, ss3):
    c = lax.axis_index("c")
    s = lax.axis_index("s")
    ibufs = (i0, i1, i2, i3)
    isems = (is0, is1, is2, is3)
    ssems = (ss0, ss1, ss2, ss3)
    one16 = jnp.full((16,), 1.0, dtype=jnp.float32)
    zero16 = jnp.zeros((16,), dtype=jnp.float32)
    for i in range(_CHD // 16):
        ones_v[pl.ds(i * 16, 16)] = one16

    def fill0(i, _):
        stg_v[pl.ds(i * 16, 16)] = zero16
        return ()

    lax.fori_loop(0, 1008 // 16, fill0, ())

    @pl.when(s < 10)
    def _():
        pltpu.sync_copy(stg_v.at[pl.ds(0, 1000)],
                        deg_sh.at[pl.ds(s * 1000, 1000)])

    @pl.when(s == 10)
    def _():
        pltpu.sync_copy(stg_v.at[pl.ds(0, _NPADROW)],
                        deg_sh.at[pl.ds(_N, _NPADROW)])

    plsc.subcore_barrier()
    base = (c * _NS + s) * _EPTP

    def i_start(ch, b):
        pltpu.async_copy(dst_hbm.at[pl.ds(base + ch * _CHD, _CHD)],
                         ibufs[b], isems[b])

    def i_wait(b):
        pltpu.make_async_copy(dst_hbm.at[pl.ds(0, _CHD)],
                              ibufs[b], isems[b]).wait()

    def sc_start(b):
        pltpu.async_copy(ones_v, deg_sh.at[ibufs[b]], ssems[b], add=True)

    def sc_wait(b):
        pltpu.make_async_copy(ones_v, deg_sh.at[pl.ds(0, _CHD)],
                              ssems[b]).wait()

    for b in range(4):
        i_start(b, b)

    def body(j, _):
        e = 4 * j
        for b in range(4):
            i_wait(b)
            sc_start(b)
        for b in range(4):
            sc_wait(b)

            @pl.when(e + 4 + b < _NCHD)
            def _():
                i_start(e + 4 + b, b)

        return ()

    lax.fori_loop(0, _NCHD // 4, body, ())
    for b in range(_NCHD % 4):
        i_wait(b)
        sc_start(b)
    for b in range(_NCHD % 4):
        sc_wait(b)
    plsc.subcore_barrier()
    @pl.when(s < 10)
    def _():
        pltpu.sync_copy(deg_sh.at[pl.ds(s * 1000, 1000)],
                        stg_v.at[pl.ds(0, 1000)])
        pltpu.sync_copy(stg_v.at[pl.ds(0, 1000)],
                        deg_hbm.at[pl.ds(c * _N + s * 1000, 1000)])


@functools.partial(
    pl.kernel,
    out_type=jax.ShapeDtypeStruct((_NC * _N, _D), jnp.float32),
    mesh=_MESH,
    scratch_types=[
        pltpu.VMEM((_EPTP,), jnp.int32),
    ] + [pltpu.VMEM((_CHP,), jnp.int32) for _ in range(_DEPTH)]
      + [pltpu.VMEM((_CHP, _D), jnp.float32) for _ in range(_DEPTH)]
      + [pltpu.VMEM_SHARED((_N + _NPADROW, _D), jnp.float32)]
      + [pltpu.SemaphoreType.DMA for _ in range(3 * _DEPTH)],
)
def _edge_acc_kernel(y_hbm, src_hbm, dst_hbm, acc_hbm, sidx, *bufs):
    ibufs = bufs[:_DEPTH]
    rbufs = bufs[_DEPTH:2 * _DEPTH]
    acc_sh = bufs[2 * _DEPTH]
    gsems = bufs[2 * _DEPTH + 1:2 * _DEPTH + 1 + _DEPTH]
    ssems = bufs[2 * _DEPTH + 1 + _DEPTH:2 * _DEPTH + 1 + 2 * _DEPTH]
    isems = bufs[2 * _DEPTH + 1 + 2 * _DEPTH:]
    rows0 = rbufs[0]
    c = lax.axis_index("c")
    s = lax.axis_index("s")
    base = (c * _NS + s) * _EPTP
    pltpu.sync_copy(src_hbm.at[pl.ds(base, _EPTP)], sidx)
    zero16 = jnp.zeros((16,), dtype=jnp.float32)

    def fill0(j, _):
        for i in range(_D // 16):
            rows0[j, pl.ds(i * 16, 16)] = zero16
        return ()

    lax.fori_loop(0, _CHW, fill0, ())

    for k in range(_NCHW // _NS + 1):
        @pl.when(s + _NS * k < _NCHW)
        def _():
            pltpu.sync_copy(rows0.at[pl.ds(0, _CHW), :],
                            acc_sh.at[pl.ds((s + _NS * k) * _CHW, _CHW), :])

    plsc.subcore_barrier()

    def g_start(ch, b):
        pltpu.async_copy(y_hbm.at[sidx.at[pl.ds(ch * _CHP, _CHP)]],
                         rbufs[b], gsems[b])

    def g_wait(b):
        pltpu.make_async_copy(y_hbm.at[pl.ds(0, _CHP), :],
                              rbufs[b], gsems[b]).wait()

    def sc_start(b):
        pltpu.async_copy(rbufs[b], acc_sh.at[ibufs[b]], ssems[b], add=True)

    def sc_wait(b):
        pltpu.make_async_copy(rbufs[b], acc_sh.at[pl.ds(0, _CHP), :],
                              ssems[b]).wait()

    def i_start(ch, b):
        pltpu.async_copy(dst_hbm.at[pl.ds(base + ch * _CHP, _CHP)],
                         ibufs[b], isems[b])

    def i_wait(b):
        pltpu.make_async_copy(dst_hbm.at[pl.ds(0, _CHP)],
                              ibufs[b], isems[b]).wait()

    for b in range(_DEPTH):
        i_start(b, b)
        g_start(b, b)

    def body(j, _):
        e = _DEPTH * j
        for b in range(_DEPTH):
            g_wait(b)
            i_wait(b)
            sc_start(b)
        for b in range(_DEPTH):
            sc_wait(b)

            @pl.when(e + _DEPTH + b < _NCHP)
            def _():
                i_start(e + _DEPTH + b, b)
                g_start(e + _DEPTH + b, b)

        return ()

    lax.fori_loop(0, _NCHP // _DEPTH, body, ())
    plsc.subcore_barrier()
    def wb_rd(k, b):
        pltpu.async_copy(acc_sh.at[pl.ds((s + _NS * k) * _CHW, _CHW), :],
                         rbufs[b].at[pl.ds(0, _CHW), :], gsems[b])

    def wb_rd_wait(b):
        pltpu.make_async_copy(acc_sh.at[pl.ds(0, _CHW), :],
                              rbufs[b].at[pl.ds(0, _CHW), :], gsems[b]).wait()

    def wb_wr(k, b):
        pltpu.async_copy(rbufs[b].at[pl.ds(0, _CHW), :],
                         acc_hbm.at[pl.ds(c * _N + (s + _NS * k) * _CHW,
                                          _CHW), :], ssems[b])

    def wb_wr_wait(b):
        pltpu.make_async_copy(rbufs[b].at[pl.ds(0, _CHW), :],
                              acc_hbm.at[pl.ds(0, _CHW), :], ssems[b]).wait()

    _KMAX = _NCHW // _NS + 1

    def _valid(k):
        return s + _NS * k < _NCHW

    @pl.when(_valid(0))
    def _():
        wb_rd(0, 0)

    for k in range(_KMAX):
        b = k % 2

        @pl.when(_valid(k))
        def _(k=k, b=b):
            wb_rd_wait(b)
            wb_wr(k, b)

        if k + 1 < _KMAX:
            @pl.when(_valid(k + 1))
            def _(k=k, b=b):
                if k >= 1:
                    wb_wr_wait(1 - b)
                wb_rd(k + 1, 1 - b)

    for k in range(_KMAX):
        @pl.when(_valid(k) & ~_valid(k + 2))
        def _(k=k):
            wb_wr_wait(k % 2)


def _tc_mm_body(x_ref, w_ref, o_ref):
    o_ref[...] = jnp.dot(x_ref[...], w_ref[...],
                         preferred_element_type=jnp.float32)


def _tc_scale_body(degt_ref, xw_ref, dinv_ref, y_ref):
    deg = degt_ref[:, 0:1] + degt_ref[:, 1:2] + 1.0
    dinv = lax.rsqrt(deg)
    dinv_ref[...] = dinv
    y_ref[...] = xw_ref[...] * dinv


def _tc_mid_body(acc_ref, y_ref, dinv_ref, w_ref, b_ref, y2_ref):
    dinv = dinv_ref[...]
    h = dinv * (acc_ref[0:_N, :] + acc_ref[_N:, :] + y_ref[...]) + b_ref[...]
    hw = jnp.dot(h, w_ref[...], preferred_element_type=jnp.float32)
    y2_ref[...] = hw * dinv


def _tc_tail_body(acc_ref, y_ref, dinv_ref, b_ref, o_ref):
    dinv = dinv_ref[...]
    o_ref[...] = dinv * (acc_ref[0:_N, :] + acc_ref[_N:, :] + y_ref[...]) + b_ref[...]


def kernel(x, edge_index, W1, b1, W2, b2):
    src = edge_index[0]
    dst = edge_index[1]

    iw = jnp.arange(_NW, dtype=jnp.int32)[:, None]
    ip = jnp.arange(_PAD, dtype=jnp.int32)[None, :]
    pad_s = (iw * 313 + ip * 41) % _N
    pad_d = _N + (iw * 8 + ip) % _NPADROW
    src_pad = jnp.concatenate([src.reshape(_NW, _EPT), pad_s], axis=1).reshape(-1)
    dst_pad = jnp.concatenate([dst.reshape(_NW, _EPT), pad_d], axis=1).reshape(-1)

    degp = _deg_kernel(dst_pad)
    degt = jnp.transpose(degp.reshape(_NC, _N))

    xw1 = pl.pallas_call(
        _tc_mm_body,
        out_shape=jax.ShapeDtypeStruct((_N, _D), jnp.float32),
    )(x, W1)

    dinv, y1 = pl.pallas_call(
        _tc_scale_body,
        out_shape=(
            jax.ShapeDtypeStruct((_N, 1), jnp.float32),
            jax.ShapeDtypeStruct((_N, _D), jnp.float32),
        ),
    )(degt, xw1)

    acc1 = _edge_acc_kernel(y1, src_pad, dst_pad)

    y2 = pl.pallas_call(
        _tc_mid_body,
        out_shape=jax.ShapeDtypeStruct((_N, _D), jnp.float32),
    )(acc1, y1, dinv, W2, b1.reshape(1, _D))

    acc2 = _edge_acc_kernel(y2, src_pad, dst_pad)

    out = pl.pallas_call(
        _tc_tail_body,
        out_shape=jax.ShapeDtypeStruct((_N, _D), jnp.float32),
    )(acc2, y2, dinv, b2.reshape(1, _D))
    return out


# speedup vs baseline: 33.6729x; 1.0078x over previous
import functools

import jax
import jax.numpy as jnp
from jax import lax
from jax.experimental import pallas as pl
from jax.experimental.pallas import tpu as pltpu
from jax.experimental.pallas import tpu_sc as plsc

_N = 10000
_D = 128
_E = 320000
_NC = 2
_NS = 16
_NW = _NC * _NS
_EPT = _E // _NW
_CHW = 40
_NCHW = _N // _CHW
_CHD = 112
_NCHD = 10080 // _CHD
_CHP = 56
_EPTP = 10080
_PAD = _EPTP - _EPT
_NCHP = _EPTP // _CHP
_DEPTH = 5
_NPADROW = 256

_MESH = plsc.VectorSubcoreMesh(
    core_axis_name="c", subcore_axis_name="s", num_cores=_NC, num_subcores=_NS
)


@functools.partial(
    pl.kernel,
    out_type=jax.ShapeDtypeStruct((_NC * _N,), jnp.float32),
    mesh=_MESH,
    scratch_types=[
        pltpu.VMEM((_CHD,), jnp.int32),
        pltpu.VMEM((_CHD,), jnp.int32),
        pltpu.VMEM((_CHD,), jnp.int32),
        pltpu.VMEM((_CHD,), jnp.int32),
        pltpu.VMEM((_CHD,), jnp.float32),
        pltpu.VMEM((1008,), jnp.float32),
        pltpu.VMEM_SHARED((_N + _NPADROW,), jnp.float32),
        pltpu.SemaphoreType.DMA,
        pltpu.SemaphoreType.DMA,
        pltpu.SemaphoreType.DMA,
        pltpu.SemaphoreType.DMA,
        pltpu.SemaphoreType.DMA,
        pltpu.SemaphoreType.DMA,
        pltpu.SemaphoreType.DMA,
        pltpu.SemaphoreType.DMA,
    ],
)
def _deg_kernel(dst_hbm, deg_hbm, i0, i1, i2, i3, ones_v, stg_v, deg_sh,
                is0, is1, is2, is3, ss0, ss1, ss2, ss3):
    c = lax.axis_index("c")
    s = lax.axis_index("s")
    ibufs = (i0, i1, i2, i3)
    isems = (is0, is1, is2, is3)
    ssems = (ss0, ss1, ss2, ss3)
    one16 = jnp.full((16,), 1.0, dtype=jnp.float32)
    zero16 = jnp.zeros((16,), dtype=jnp.float32)
    for i in range(_CHD // 16):
        ones_v[pl.ds(i * 16, 16)] = one16

    def fill0(i, _):
        stg_v[pl.ds(i * 16, 16)] = zero16
        return ()

    lax.fori_loop(0, 1008 // 16, fill0, ())

    @pl.when(s < 10)
    def _():
        pltpu.sync_copy(stg_v.at[pl.ds(0, 1000)],
                        deg_sh.at[pl.ds(s * 1000, 1000)])

    @pl.when(s == 10)
    def _():
        pltpu.sync_copy(stg_v.at[pl.ds(0, _NPADROW)],
                        deg_sh.at[pl.ds(_N, _NPADROW)])

    plsc.subcore_barrier()
    base = (c * _NS + s) * _EPTP

    def i_start(ch, b):
        pltpu.async_copy(dst_hbm.at[pl.ds(base + ch * _CHD, _CHD)],
                         ibufs[b], isems[b])

    def i_wait(b):
        pltpu.make_async_copy(dst_hbm.at[pl.ds(0, _CHD)],
                              ibufs[b], isems[b]).wait()

    def sc_start(b):
        pltpu.async_copy(ones_v, deg_sh.at[ibufs[b]], ssems[b], add=True)

    def sc_wait(b):
        pltpu.make_async_copy(ones_v, deg_sh.at[pl.ds(0, _CHD)],
                              ssems[b]).wait()

    for b in range(4):
        i_start(b, b)

    def body(j, _):
        e = 4 * j
        for b in range(4):
            i_wait(b)
            sc_start(b)
        for b in range(4):
            sc_wait(b)

            @pl.when(e + 4 + b < _NCHD)
            def _():
                i_start(e + 4 + b, b)

        return ()

    lax.fori_loop(0, _NCHD // 4, body, ())
    for b in range(_NCHD % 4):
        i_wait(b)
        sc_start(b)
    for b in range(_NCHD % 4):
        sc_wait(b)
    plsc.subcore_barrier()
    @pl.when(s < 10)
    def _():
        pltpu.sync_copy(deg_sh.at[pl.ds(s * 1000, 1000)],
                        stg_v.at[pl.ds(0, 1000)])
        pltpu.sync_copy(stg_v.at[pl.ds(0, 1000)],
                        deg_hbm.at[pl.ds(c * _N + s * 1000, 1000)])


@functools.partial(
    pl.kernel,
    out_type=jax.ShapeDtypeStruct((_NC * _N, _D), jnp.float32),
    mesh=_MESH,
    scratch_types=[
        pltpu.VMEM((_EPTP,), jnp.int32),
    ] + [pltpu.VMEM((_CHP,), jnp.int32) for _ in range(_DEPTH)]
      + [pltpu.VMEM((_CHP, _D), jnp.float32) for _ in range(_DEPTH)]
      + [pltpu.VMEM_SHARED((_N + _NPADROW, _D), jnp.float32)]
      + [pltpu.SemaphoreType.DMA for _ in range(3 * _DEPTH)],
)
def _edge_acc_kernel(y_hbm, src_hbm, dst_hbm, acc_hbm, sidx, *bufs):
    ibufs = bufs[:_DEPTH]
    rbufs = bufs[_DEPTH:2 * _DEPTH]
    acc_sh = bufs[2 * _DEPTH]
    gsems = bufs[2 * _DEPTH + 1:2 * _DEPTH + 1 + _DEPTH]
    ssems = bufs[2 * _DEPTH + 1 + _DEPTH:2 * _DEPTH + 1 + 2 * _DEPTH]
    isems = bufs[2 * _DEPTH + 1 + 2 * _DEPTH:]
    rows0 = rbufs[0]
    c = lax.axis_index("c")
    s = lax.axis_index("s")
    base = (c * _NS + s) * _EPTP
    pltpu.async_copy(src_hbm.at[pl.ds(base, _EPTP)], sidx, gsems[0])
    zero16 = jnp.zeros((16,), dtype=jnp.float32)

    def fill0(j, _):
        for i in range(_D // 16):
            rows0[j, pl.ds(i * 16, 16)] = zero16
        return ()

    lax.fori_loop(0, _CHW, fill0, ())

    for k in range(_NCHW // _NS + 1):
        @pl.when(s + _NS * k < _NCHW)
        def _():
            pltpu.sync_copy(rows0.at[pl.ds(0, _CHW), :],
                            acc_sh.at[pl.ds((s + _NS * k) * _CHW, _CHW), :])

    pltpu.make_async_copy(src_hbm.at[pl.ds(0, _EPTP)], sidx, gsems[0]).wait()
    plsc.subcore_barrier()

    def g_start(ch, b):
        pltpu.async_copy(y_hbm.at[sidx.at[pl.ds(ch * _CHP, _CHP)]],
                         rbufs[b], gsems[b])

    def g_wait(b):
        pltpu.make_async_copy(y_hbm.at[pl.ds(0, _CHP), :],
                              rbufs[b], gsems[b]).wait()

    def sc_start(b):
        pltpu.async_copy(rbufs[b], acc_sh.at[ibufs[b]], ssems[b], add=True)

    def sc_wait(b):
        pltpu.make_async_copy(rbufs[b], acc_sh.at[pl.ds(0, _CHP), :],
                              ssems[b]).wait()

    def i_start(ch, b):
        pltpu.async_copy(dst_hbm.at[pl.ds(base + ch * _CHP, _CHP)],
                         ibufs[b], isems[b])

    def i_wait(b):
        pltpu.make_async_copy(dst_hbm.at[pl.ds(0, _CHP)],
                              ibufs[b], isems[b]).wait()

    for b in range(_DEPTH):
        i_start(b, b)
        g_start(b, b)

    def body(j, _):
        e = _DEPTH * j
        for b in range(_DEPTH):
            g_wait(b)
            i_wait(b)
            sc_start(b)
        for b in range(_DEPTH):
            sc_wait(b)

            @pl.when(e + _DEPTH + b < _NCHP)
            def _():
                i_start(e + _DEPTH + b, b)
                g_start(e + _DEPTH + b, b)

        return ()

    lax.fori_loop(0, _NCHP // _DEPTH, body, ())
    plsc.subcore_barrier()
    def wb_rd(k, b):
        pltpu.async_copy(acc_sh.at[pl.ds((s + _NS * k) * _CHW, _CHW), :],
                         rbufs[b].at[pl.ds(0, _CHW), :], gsems[b])

    def wb_rd_wait(b):
        pltpu.make_async_copy(acc_sh.at[pl.ds(0, _CHW), :],
                              rbufs[b].at[pl.ds(0, _CHW), :], gsems[b]).wait()

    def wb_wr(k, b):
        pltpu.async_copy(rbufs[b].at[pl.ds(0, _CHW), :],
                         acc_hbm.at[pl.ds(c * _N + (s + _NS * k) * _CHW,
                                          _CHW), :], ssems[b])

    def wb_wr_wait(b):
        pltpu.make_async_copy(rbufs[b].at[pl.ds(0, _CHW), :],
                              acc_hbm.at[pl.ds(0, _CHW), :], ssems[b]).wait()

    _KMAX = _NCHW // _NS + 1

    def _valid(k):
        return s + _NS * k < _NCHW

    @pl.when(_valid(0))
    def _():
        wb_rd(0, 0)

    for k in range(_KMAX):
        b = k % 2

        @pl.when(_valid(k))
        def _(k=k, b=b):
            wb_rd_wait(b)
            wb_wr(k, b)

        if k + 1 < _KMAX:
            @pl.when(_valid(k + 1))
            def _(k=k, b=b):
                if k >= 1:
                    wb_wr_wait(1 - b)
                wb_rd(k + 1, 1 - b)

    for k in range(_KMAX):
        @pl.when(_valid(k) & ~_valid(k + 2))
        def _(k=k):
            wb_wr_wait(k % 2)


def _tc_mm_body(x_ref, w_ref, o_ref):
    o_ref[...] = jnp.dot(x_ref[...], w_ref[...],
                         preferred_element_type=jnp.float32)


def _tc_scale_body(degt_ref, xw_ref, dinv_ref, y_ref):
    deg = degt_ref[:, 0:1] + degt_ref[:, 1:2] + 1.0
    dinv = lax.rsqrt(deg)
    dinv_ref[...] = dinv
    y_ref[...] = xw_ref[...] * dinv


def _tc_mid_body(acc_ref, y_ref, dinv_ref, w_ref, b_ref, y2_ref):
    dinv = dinv_ref[...]
    h = dinv * (acc_ref[0:_N, :] + acc_ref[_N:, :] + y_ref[...]) + b_ref[...]
    hw = jnp.dot(h, w_ref[...], preferred_element_type=jnp.float32)
    y2_ref[...] = hw * dinv


def _tc_tail_body(acc_ref, y_ref, dinv_ref, b_ref, o_ref):
    dinv = dinv_ref[...]
    o_ref[...] = dinv * (acc_ref[0:_N, :] + acc_ref[_N:, :] + y_ref[...]) + b_ref[...]


def kernel(x, edge_index, W1, b1, W2, b2):
    src = edge_index[0]
    dst = edge_index[1]

    iw = jnp.arange(_NW, dtype=jnp.int32)[:, None]
    ip = jnp.arange(_PAD, dtype=jnp.int32)[None, :]
    pad_s = (iw * 313 + ip * 41) % _N
    pad_d = _N + (iw * 8 + ip) % _NPADROW
    src_pad = jnp.concatenate([src.reshape(_NW, _EPT), pad_s], axis=1).reshape(-1)
    dst_pad = jnp.concatenate([dst.reshape(_NW, _EPT), pad_d], axis=1).reshape(-1)

    degp = _deg_kernel(dst_pad)
    degt = jnp.transpose(degp.reshape(_NC, _N))

    xw1 = pl.pallas_call(
        _tc_mm_body,
        out_shape=jax.ShapeDtypeStruct((_N, _D), jnp.float32),
    )(x, W1)

    dinv, y1 = pl.pallas_call(
        _tc_scale_body,
        out_shape=(
            jax.ShapeDtypeStruct((_N, 1), jnp.float32),
            jax.ShapeDtypeStruct((_N, _D), jnp.float32),
        ),
    )(degt, xw1)

    acc1 = _edge_acc_kernel(y1, src_pad, dst_pad)

    y2 = pl.pallas_call(
        _tc_mid_body,
        out_shape=jax.ShapeDtypeStruct((_N, _D), jnp.float32),
    )(acc1, y1, dinv, W2, b1.reshape(1, _D))

    acc2 = _edge_acc_kernel(y2, src_pad, dst_pad)

    out = pl.pallas_call(
        _tc_tail_body,
        out_shape=jax.ShapeDtypeStruct((_N, _D), jnp.float32),
    )(acc2, y2, dinv, b2.reshape(1, _D))
    return out
